# baseline (speedup 1.0000x reference)
import functools

import jax
import jax.numpy as jnp
from jax import lax
from jax.experimental import pallas as pl
from jax.experimental.pallas import tpu as pltpu
from jax.experimental.pallas import tpu_sc as plsc

L = 2
D = 768
H = 12
DH = 64
DFF = 1024
G = 4
EPG = 4
E = 16
TOPK = 2
CAP = 320
NCLS = 10
S1 = 2049
SP = 2304
NB = SP // 128
SA = 2176
NBA = SA // 128
NBUF = E * CAP + 8
TRASH = E * CAP
NW = 32
CHUNK = SP // NW

_F32 = jnp.float32


def _dot(a, b):
    return jnp.dot(a, b, preferred_element_type=_F32)


def _dot32(a, b):
    return jnp.dot(a, b, preferred_element_type=_F32)


def _dotbf(a, b):
    return jnp.dot(a.astype(jnp.bfloat16), b, preferred_element_type=_F32)


def _embed_kernel(xp_ref, w_ref, tab_ref, out_ref):
    out_ref[...] = _dot(xp_ref[...], w_ref[...]) + tab_ref[...]


def _embed(xp, w_in, table):
    return pl.pallas_call(
        _embed_kernel,
        grid=(NBA,),
        in_specs=[
            pl.BlockSpec((128, D), lambda i: (i, 0)),
            pl.BlockSpec((D, D), lambda i: (0, 0)),
            pl.BlockSpec((128, D), lambda i: (i, 0)),
        ],
        out_specs=pl.BlockSpec((128, D), lambda i: (i, 0)),
        out_shape=jax.ShapeDtypeStruct((SP, D), _F32),
    )(xp.astype(jnp.bfloat16), w_in.astype(jnp.bfloat16), table)


def _ln_rows(x, g, b):
    m = jnp.mean(x, axis=1, keepdims=True)
    c = x - m
    v = jnp.mean(c * c, axis=1, keepdims=True)
    return c * lax.rsqrt(v + 1e-5) * g + b


def _qkv_kernel(h_ref, g_ref, b_ref, w_ref, bias_ref, out_ref):
    t = _ln_rows(h_ref[...], g_ref[...], b_ref[...])
    out_ref[...] = (_dotbf(t, w_ref[...]) + bias_ref[...]).astype(jnp.bfloat16)


def _qkv(h, ln_g, ln_b, wqkv, bqkv):
    return pl.pallas_call(
        _qkv_kernel,
        grid=(NBA,),
        in_specs=[
            pl.BlockSpec((128, D), lambda i: (i, 0)),
            pl.BlockSpec((1, D), lambda i: (0, 0)),
            pl.BlockSpec((1, D), lambda i: (0, 0)),
            pl.BlockSpec((D, 3 * D), lambda i: (0, 0)),
            pl.BlockSpec((1, 3 * D), lambda i: (0, 0)),
        ],
        out_specs=pl.BlockSpec((128, 3 * D), lambda i: (i, 0)),
        out_shape=jax.ShapeDtypeStruct((SP, 3 * D), jnp.bfloat16),
    )(h, ln_g.reshape(1, D), ln_b.reshape(1, D),
      wqkv.astype(jnp.bfloat16), bqkv.reshape(1, 3 * D))


def _attn_kernel(q_ref, k_ref, v_ref, out_ref):
    col = lax.broadcasted_iota(jnp.int32, (128, SA), 1)
    for off in (0, DH):
        q = q_ref[:, off:off + DH]
        k = k_ref[:, off:off + DH]
        s = lax.dot_general(q, k, (((1,), (1,)), ((), ())),
                            preferred_element_type=_F32) * (1.0 / (DH ** 0.5))
        s = jnp.where(col < S1, s, -1e30)
        m = jnp.max(s, axis=1, keepdims=True)
        p = jnp.exp((s - m).astype(jnp.bfloat16))
        ones = jnp.ones((SA, 1), jnp.bfloat16)
        r = _dot(p, ones)
        pv = _dot(p, v_ref[:, off:off + DH])
        out_ref[:, off:off + DH] = pv / r


def _attention(qkv, nqb=NBA, out_rows=SP):
    return pl.pallas_call(
        _attn_kernel,
        grid=(H // 2, nqb),
        in_specs=[
            pl.BlockSpec((128, 2 * DH), lambda h, i: (i, h)),
            pl.BlockSpec((SA, 2 * DH), lambda h, i: (0, H // 2 + h)),
            pl.BlockSpec((SA, 2 * DH), lambda h, i: (0, H + h)),
        ],
        out_specs=pl.BlockSpec((128, 2 * DH), lambda h, i: (i, h)),
        out_shape=jax.ShapeDtypeStruct((out_rows, D), _F32),
    )(qkv, qkv, qkv)


def _proj_kernel(o_ref, h_ref, w_ref, b_ref, out_ref):
    out_ref[...] = h_ref[...] + _dotbf(o_ref[...], w_ref[...]) + b_ref[...]


def _proj_residual(o, h, wo, bo, nb=NBA, out_rows=SP):
    return pl.pallas_call(
        _proj_kernel,
        grid=(nb,),
        in_specs=[
            pl.BlockSpec((128, D), lambda i: (i, 0)),
            pl.BlockSpec((128, D), lambda i: (i, 0)),
            pl.BlockSpec((D, D), lambda i: (0, 0)),
            pl.BlockSpec((1, D), lambda i: (0, 0)),
        ],
        out_specs=pl.BlockSpec((128, D), lambda i: (i, 0)),
        out_shape=jax.ShapeDtypeStruct((out_rows, D), _F32),
    )(o, h, wo.astype(jnp.bfloat16), bo.reshape(1, D))


def _router_kernel(rb, h_ref, g_ref, b_ref, wcat_ref,
                   xf_ref, d1_ref, d2_ref, s1_ref, s2_ref, cnt_ref):
    blk = pl.program_id(0)

    @pl.when(blk == 0)
    def _init():
        cnt_ref[...] = jnp.zeros_like(cnt_ref)

    xf = _ln_rows(h_ref[...], g_ref[...], b_ref[...])
    xf_ref[...] = xf

    z = _dot32(xf, wcat_ref[...])
    zg = z[:, 0:G]
    ze = z[:, G:G + E]

    zg = zg - jnp.max(zg, axis=1, keepdims=True)
    eg = jnp.exp(zg)
    gp = eg / jnp.sum(eg, axis=1, keepdims=True)

    grow = lax.broadcasted_iota(jnp.int32, (G, E), 0)
    gcol = lax.broadcasted_iota(jnp.int32, (G, E), 1)
    expand = (gcol // EPG == grow).astype(_F32)
    gpf = _dot32(gp, expand)

    ze = ze - jnp.max(ze, axis=1, keepdims=True)
    ee = jnp.exp(ze)
    mr = lax.broadcasted_iota(jnp.int32, (E, E), 0)
    mc = lax.broadcasted_iota(jnp.int32, (E, E), 1)
    gmask = (mr // EPG == mc // EPG).astype(_F32)
    seg = _dot32(ee, gmask)
    probs = gpf * ee / seg

    lane = lax.broadcasted_iota(jnp.int32, (rb, E), 1)
    m1 = jnp.max(probs, axis=1, keepdims=True)
    idx1 = jnp.min(jnp.where(probs == m1, lane, E), axis=1, keepdims=True)
    p2 = jnp.where(lane == idx1, -1.0, probs)
    m2 = jnp.max(p2, axis=1, keepdims=True)
    idx2 = jnp.min(jnp.where(p2 == m2, lane, E), axis=1, keepdims=True)
    denom = m1 + m2 + 1e-9
    g1 = m1 / denom
    g2 = m2 / denom

    row = lax.broadcasted_iota(jnp.int32, (rb, 1), 0)
    valid = (blk * rb + row) < S1
    bf = jnp.bfloat16
    oh1 = ((lane == idx1) & valid).astype(bf)
    oh2 = ((lane == idx2) & valid).astype(bf)
    oh = oh1 + oh2

    tr = lax.broadcasted_iota(jnp.int32, (rb, rb), 0)
    tc = lax.broadcasted_iota(jnp.int32, (rb, rb), 1)
    ltri = (tc < tr).astype(bf)
    carry = cnt_ref[0:1, 0:E]
    excl = _dot(ltri, oh) + carry
    cnt_ref[0:1, 0:E] = carry + jnp.sum(oh.astype(_F32), axis=0, keepdims=True)

    oh1f = oh1.astype(_F32)
    oh2f = oh2.astype(_F32)
    r1 = jnp.sum(oh1f * excl, axis=1, keepdims=True)
    r2 = jnp.sum(oh2f * excl, axis=1, keepdims=True)
    kept1 = valid & (r1 < CAP)
    kept2 = valid & (r2 < CAP)
    d1_ref[...] = jnp.where(kept1, idx1 * CAP + r1.astype(jnp.int32), TRASH)
    d2_ref[...] = jnp.where(kept2, idx2 * CAP + r2.astype(jnp.int32), TRASH)
    s1_ref[...] = jnp.where(kept1, g1, 0.0)
    s2_ref[...] = jnp.where(kept2, g2, 0.0)


def _router(h, ln_g, ln_b, wcat, nb=3, rb=768):
    return pl.pallas_call(
        functools.partial(_router_kernel, rb),
        grid=(nb,),
        in_specs=[
            pl.BlockSpec((rb, D), lambda i: (i, 0)),
            pl.BlockSpec((1, D), lambda i: (0, 0)),
            pl.BlockSpec((1, D), lambda i: (0, 0)),
            pl.BlockSpec((D, G + E), lambda i: (0, 0)),
        ],
        out_specs=[
            pl.BlockSpec((rb, D), lambda i: (i, 0)),
            pl.BlockSpec((rb, 1), lambda i: (i, 0)),
            pl.BlockSpec((rb, 1), lambda i: (i, 0)),
            pl.BlockSpec((rb, 1), lambda i: (i, 0)),
            pl.BlockSpec((rb, 1), lambda i: (i, 0)),
        ],
        out_shape=[
            jax.ShapeDtypeStruct((nb * rb, D), _F32),
            jax.ShapeDtypeStruct((nb * rb, 1), jnp.int32),
            jax.ShapeDtypeStruct((nb * rb, 1), jnp.int32),
            jax.ShapeDtypeStruct((nb * rb, 1), _F32),
            jax.ShapeDtypeStruct((nb * rb, 1), _F32),
        ],
        scratch_shapes=[pltpu.VMEM((8, 128), _F32)],
    )(h, ln_g.reshape(1, D), ln_b.reshape(1, D), wcat)


def _sc_dispatch(xf, d1, d2):
    mesh = plsc.VectorSubcoreMesh(core_axis_name="c", subcore_axis_name="s",
                                  num_cores=2, num_subcores=16)

    @functools.partial(
        pl.kernel,
        out_type=jax.ShapeDtypeStruct((NBUF, D), _F32),
        mesh=mesh,
        scratch_types=[
            pltpu.VMEM((CHUNK,), jnp.int32),
            pltpu.VMEM((CHUNK,), jnp.int32),
            pltpu.VMEM((CHUNK, D), _F32),
            pltpu.SemaphoreType.DMA,
            pltpu.SemaphoreType.DMA,
            pltpu.SemaphoreType.DMA,
        ],
    )
    def k(xf_hbm, d1_hbm, d2_hbm, buf_hbm, idx1_v, idx2_v, rows_v,
          semr, sem1, sem2):
        wid = lax.axis_index("s") * 2 + lax.axis_index("c")
        base = wid * CHUNK
        cr = pltpu.async_copy(xf_hbm.at[pl.ds(base, CHUNK)], rows_v, semr)
        pltpu.sync_copy(d1_hbm.at[pl.ds(base, CHUNK)], idx1_v)
        pltpu.sync_copy(d2_hbm.at[pl.ds(base, CHUNK)], idx2_v)
        cr.wait()
        c1 = pltpu.async_copy(rows_v, buf_hbm.at[idx1_v], sem1)
        c2 = pltpu.async_copy(rows_v, buf_hbm.at[idx2_v], sem2)
        c1.wait()
        c2.wait()

    return k(xf, d1, d2)


def _sc_gather(y, d1, d2):
    mesh = plsc.VectorSubcoreMesh(core_axis_name="c", subcore_axis_name="s",
                                  num_cores=2, num_subcores=16)

    @functools.partial(
        pl.kernel,
        out_type=(jax.ShapeDtypeStruct((SP, D), _F32),
                  jax.ShapeDtypeStruct((SP, D), _F32)),
        mesh=mesh,
        scratch_types=[
            pltpu.VMEM((CHUNK,), jnp.int32),
            pltpu.VMEM((CHUNK,), jnp.int32),
            pltpu.VMEM((CHUNK, D), _F32),
            pltpu.VMEM((CHUNK, D), _F32),
            pltpu.SemaphoreType.DMA,
            pltpu.SemaphoreType.DMA,
            pltpu.SemaphoreType.DMA,
            pltpu.SemaphoreType.DMA,
        ],
    )
    def k(y_hbm, d1_hbm, d2_hbm, g1_hbm, g2_hbm, idx1_v, idx2_v,
          rows1_v, rows2_v, sem1, sem2, semw1, semw2):
        wid = lax.axis_index("s") * 2 + lax.axis_index("c")
        base = wid * CHUNK
        pltpu.sync_copy(d1_hbm.at[pl.ds(base, CHUNK)], idx1_v)
        pltpu.sync_copy(d2_hbm.at[pl.ds(base, CHUNK)], idx2_v)
        c1 = pltpu.async_copy(y_hbm.at[idx1_v], rows1_v, sem1)
        c2 = pltpu.async_copy(y_hbm.at[idx2_v], rows2_v, sem2)
        c1.wait()
        w1 = pltpu.async_copy(rows1_v, g1_hbm.at[pl.ds(base, CHUNK)], semw1)
        c2.wait()
        w2 = pltpu.async_copy(rows2_v, g2_hbm.at[pl.ds(base, CHUNK)], semw2)
        w1.wait()
        w2.wait()

    return k(y, d1, d2)


def _ffn_kernel(buf_ref, w1_ref, b1_ref, w2_ref, b2_ref, y_ref):
    a = _dotbf(buf_ref[...], w1_ref[0]) + b1_ref[0]
    hgelu = jax.nn.gelu(a)
    y_ref[...] = _dotbf(hgelu, w2_ref[0]) + b2_ref[0]


def _ffn(buf, w1, b1, w2, b2):
    return pl.pallas_call(
        _ffn_kernel,
        grid=(E,),
        in_specs=[
            pl.BlockSpec((CAP, D), lambda e: (e, 0)),
            pl.BlockSpec((1, D, DFF), lambda e: (e, 0, 0)),
            pl.BlockSpec((1, 1, DFF), lambda e: (e, 0, 0)),
            pl.BlockSpec((1, DFF, D), lambda e: (e, 0, 0)),
            pl.BlockSpec((1, 1, D), lambda e: (e, 0, 0)),
        ],
        out_specs=pl.BlockSpec((CAP, D), lambda e: (e, 0)),
        out_shape=jax.ShapeDtypeStruct((NBUF, D), _F32),
    )(buf, w1.astype(jnp.bfloat16), b1.reshape(E, 1, DFF),
      w2.astype(jnp.bfloat16), b2.reshape(E, 1, D))


def _combine_kernel(h_ref, g1_ref, g2_ref, s1_ref, s2_ref, out_ref):
    s1 = s1_ref[...]
    s2 = s2_ref[...]
    t1 = jnp.where(s1 > 0.0, s1 * g1_ref[...], 0.0)
    t2 = jnp.where(s2 > 0.0, s2 * g2_ref[...], 0.0)
    out_ref[...] = h_ref[...] + t1 + t2


def _combine(h, gg1, gg2, s1, s2):
    return pl.pallas_call(
        _combine_kernel,
        grid=(NBA,),
        in_specs=[
            pl.BlockSpec((128, D), lambda i: (i, 0)),
            pl.BlockSpec((128, D), lambda i: (i, 0)),
            pl.BlockSpec((128, D), lambda i: (i, 0)),
            pl.BlockSpec((128, 1), lambda i: (i, 0)),
            pl.BlockSpec((128, 1), lambda i: (i, 0)),
        ],
        out_specs=pl.BlockSpec((128, D), lambda i: (i, 0)),
        out_shape=jax.ShapeDtypeStruct((SP, D), _F32),
    )(h, gg1, gg2, s1, s2)


def _head_moe_kernel(eids_ref, h0_ref, xf_ref, s_ref, w1_ref, b1_ref,
                     w2_ref, b2_ref, lg_ref, lb_ref, hw_ref, hb_ref,
                     out_ref, acc_ref):
    i = pl.program_id(0)

    @pl.when(i == 0)
    def _init():
        acc_ref[...] = h0_ref[...]

    a = jax.nn.gelu(_dotbf(xf_ref[0:1, :], w1_ref[0]) + b1_ref[0])
    y0 = _dotbf(a, w2_ref[0]) + b2_ref[0]
    sv = s_ref[pl.ds(i, 1), :]
    acc_ref[0:1, :] = acc_ref[0:1, :] + sv * y0

    @pl.when(i == TOPK - 1)
    def _fin():
        rep = _ln_rows(acc_ref[0:1, :], lg_ref[...], lb_ref[...])
        out_ref[...] = _dot32(rep, hw_ref[...]) + hb_ref[...]


def _head_moe(eids, h0, xf0, svec, w1, b1, w2, b2, lnf_g, lnf_b,
              head_w, head_b):
    spec = pltpu.PrefetchScalarGridSpec(
        num_scalar_prefetch=1,
        grid=(TOPK,),
        in_specs=[
            pl.BlockSpec((8, D), lambda i, e: (0, 0)),
            pl.BlockSpec((8, D), lambda i, e: (0, 0)),
            pl.BlockSpec((TOPK, 1), lambda i, e: (0, 0)),
            pl.BlockSpec((1, D, DFF), lambda i, e: (e[i], 0, 0)),
            pl.BlockSpec((1, 1, DFF), lambda i, e: (e[i], 0, 0)),
            pl.BlockSpec((1, DFF, D), lambda i, e: (e[i], 0, 0)),
            pl.BlockSpec((1, 1, D), lambda i, e: (e[i], 0, 0)),
            pl.BlockSpec((1, D), lambda i, e: (0, 0)),
            pl.BlockSpec((1, D), lambda i, e: (0, 0)),
            pl.BlockSpec((D, NCLS), lambda i, e: (0, 0)),
            pl.BlockSpec((1, NCLS), lambda i, e: (0, 0)),
        ],
        out_specs=pl.BlockSpec((1, NCLS), lambda i, e: (0, 0)),
        scratch_shapes=[pltpu.VMEM((8, D), _F32)],
    )
    return pl.pallas_call(
        _head_moe_kernel,
        grid_spec=spec,
        out_shape=jax.ShapeDtypeStruct((1, NCLS), _F32),
    )(eids, h0, xf0, svec, w1.astype(jnp.bfloat16), b1.reshape(E, 1, DFF),
      w2.astype(jnp.bfloat16), b2.reshape(E, 1, D), lnf_g.reshape(1, D),
      lnf_b.reshape(1, D), head_w, head_b.reshape(1, NCLS))


def kernel(x, W_in, b_in, cls_token, pos_emb, ln1_g, ln1_b, Wqkv, bqkv, Wo, bo,
           ln2_g, ln2_b, Wg, We, W1, b1, W2, b2, lnf_g, lnf_b, head_W, head_b):
    xp = jnp.pad(x[0], ((1, SP - 1 - x.shape[1]), (0, 0)))
    table = jnp.concatenate([
        cls_token[0] + pos_emb[0:1],
        pos_emb[1:S1] + b_in[None, :],
        jnp.zeros((SP - S1, D), _F32),
    ], axis=0)

    h = _embed(xp, W_in, table)

    qkv = _qkv(h, ln1_g[0], ln1_b[0], Wqkv[0], bqkv[0])
    o = _attention(qkv)
    h = _proj_residual(o, h, Wo[0], bo[0])
    wcat = jnp.concatenate(
        [Wg[0], We[0].transpose(1, 0, 2).reshape(D, E)], axis=1)
    xf, d1, d2, s1, s2 = _router(h, ln2_g[0], ln2_b[0], wcat)
    d1f = d1.reshape(SP)
    d2f = d2.reshape(SP)
    buf = _sc_dispatch(xf, d1f, d2f)
    y = _ffn(buf, W1[0], b1[0], W2[0], b2[0])
    gg1, gg2 = _sc_gather(y, d1f, d2f)
    h = _combine(h, gg1, gg2, s1, s2)

    qkv = _qkv(h, ln1_g[1], ln1_b[1], Wqkv[1], bqkv[1])
    o0 = _attention(qkv, nqb=1, out_rows=128)
    h0 = _proj_residual(o0, h, Wo[1], bo[1], nb=1, out_rows=128)
    wcat = jnp.concatenate(
        [Wg[1], We[1].transpose(1, 0, 2).reshape(D, E)], axis=1)
    xf0, d1, d2, s1, s2 = _router(h0, ln2_g[1], ln2_b[1], wcat, nb=1, rb=128)
    eids = jnp.stack([d1[0, 0], d2[0, 0]]).astype(jnp.int32) // CAP
    svec = jnp.stack([s1[0, 0], s2[0, 0]]).reshape(TOPK, 1)
    return _head_moe(eids, h0, xf0, svec, W1[1], b1[1], W2[1], b2[1],
                     lnf_g, lnf_b, head_W, head_b)


# speedup vs baseline: 1.0994x; 1.0994x over previous
import functools

import jax
import jax.numpy as jnp
from jax import lax
from jax.experimental import pallas as pl
from jax.experimental.pallas import tpu as pltpu
from jax.experimental.pallas import tpu_sc as plsc

L = 2
D = 768
H = 12
DH = 64
DFF = 1024
G = 4
EPG = 4
E = 16
TOPK = 2
CAP = 320
NCLS = 10
S1 = 2049
SP = 2304
NB = SP // 128
SA = 2176
NBA = SA // 128
NBUF = E * CAP + 8
TRASH = E * CAP
NW = 32
CHUNK = SP // NW

_F32 = jnp.float32


def _dot(a, b):
    return jnp.dot(a, b, preferred_element_type=_F32)


def _dot32(a, b):
    return jnp.dot(a, b, preferred_element_type=_F32)


def _dotbf(a, b):
    return jnp.dot(a.astype(jnp.bfloat16), b, preferred_element_type=_F32)


def _embed_kernel(xp_ref, w_ref, tab_ref, out_ref):
    out_ref[...] = _dot(xp_ref[...], w_ref[...]) + tab_ref[...]


def _embed(xp, w_in, table):
    return pl.pallas_call(
        _embed_kernel,
        grid=(NBA,),
        in_specs=[
            pl.BlockSpec((128, D), lambda i: (i, 0)),
            pl.BlockSpec((D, D), lambda i: (0, 0)),
            pl.BlockSpec((128, D), lambda i: (i, 0)),
        ],
        out_specs=pl.BlockSpec((128, D), lambda i: (i, 0)),
        out_shape=jax.ShapeDtypeStruct((SP, D), _F32),
    )(xp.astype(jnp.bfloat16), w_in.astype(jnp.bfloat16), table)


def _ln_rows(x, g, b):
    m = jnp.mean(x, axis=1, keepdims=True)
    c = x - m
    v = jnp.mean(c * c, axis=1, keepdims=True)
    return c * lax.rsqrt(v + 1e-5) * g + b


def _qkv_kernel(h_ref, g_ref, b_ref, w_ref, bias_ref, out_ref):
    t = _ln_rows(h_ref[...], g_ref[...], b_ref[...])
    out_ref[...] = (_dotbf(t, w_ref[...]) + bias_ref[...]).astype(jnp.bfloat16)


def _qkv(h, ln_g, ln_b, wqkv, bqkv):
    return pl.pallas_call(
        _qkv_kernel,
        grid=(NBA,),
        in_specs=[
            pl.BlockSpec((128, D), lambda i: (i, 0)),
            pl.BlockSpec((1, D), lambda i: (0, 0)),
            pl.BlockSpec((1, D), lambda i: (0, 0)),
            pl.BlockSpec((D, 3 * D), lambda i: (0, 0)),
            pl.BlockSpec((1, 3 * D), lambda i: (0, 0)),
        ],
        out_specs=pl.BlockSpec((128, 3 * D), lambda i: (i, 0)),
        out_shape=jax.ShapeDtypeStruct((SP, 3 * D), jnp.bfloat16),
    )(h, ln_g.reshape(1, D), ln_b.reshape(1, D),
      wqkv.astype(jnp.bfloat16), bqkv.reshape(1, 3 * D))


def _attn_kernel(qb, q_ref, k_ref, v_ref, out_ref):
    col = lax.broadcasted_iota(jnp.int32, (qb, SA), 1)
    for off in (0, DH):
        q = q_ref[:, off:off + DH]
        k = k_ref[:, off:off + DH]
        s = lax.dot_general(q, k, (((1,), (1,)), ((), ())),
                            preferred_element_type=_F32) * (1.0 / (DH ** 0.5))
        s = jnp.where(col < S1, s, -1e30)
        m = jnp.max(s, axis=1, keepdims=True)
        p = jnp.exp(s - m).astype(jnp.bfloat16)
        ones = jnp.ones((SA, 1), jnp.bfloat16)
        r = _dot(p, ones)
        pv = _dot(p, v_ref[:, off:off + DH])
        out_ref[:, off:off + DH] = pv / r


def _attention(qkv, nqb=4, qb=576, out_rows=SP):
    return pl.pallas_call(
        functools.partial(_attn_kernel, qb),
        grid=(H // 2, nqb),
        in_specs=[
            pl.BlockSpec((qb, 2 * DH), lambda h, i: (i, h)),
            pl.BlockSpec((SA, 2 * DH), lambda h, i: (0, H // 2 + h)),
            pl.BlockSpec((SA, 2 * DH), lambda h, i: (0, H + h)),
        ],
        out_specs=pl.BlockSpec((qb, 2 * DH), lambda h, i: (i, h)),
        out_shape=jax.ShapeDtypeStruct((out_rows, D), _F32),
    )(qkv, qkv, qkv)


def _proj_kernel(o_ref, h_ref, w_ref, b_ref, out_ref):
    out_ref[...] = h_ref[...] + _dotbf(o_ref[...], w_ref[...]) + b_ref[...]


def _proj_residual(o, h, wo, bo, nb=NBA, out_rows=SP):
    return pl.pallas_call(
        _proj_kernel,
        grid=(nb,),
        in_specs=[
            pl.BlockSpec((128, D), lambda i: (i, 0)),
            pl.BlockSpec((128, D), lambda i: (i, 0)),
            pl.BlockSpec((D, D), lambda i: (0, 0)),
            pl.BlockSpec((1, D), lambda i: (0, 0)),
        ],
        out_specs=pl.BlockSpec((128, D), lambda i: (i, 0)),
        out_shape=jax.ShapeDtypeStruct((out_rows, D), _F32),
    )(o, h, wo.astype(jnp.bfloat16), bo.reshape(1, D))


def _router_kernel(rb, h_ref, g_ref, b_ref, wcat_ref,
                   xf_ref, d1_ref, d2_ref, s1_ref, s2_ref, cnt_ref):
    blk = pl.program_id(0)

    @pl.when(blk == 0)
    def _init():
        cnt_ref[...] = jnp.zeros_like(cnt_ref)

    xf = _ln_rows(h_ref[...], g_ref[...], b_ref[...])
    xf_ref[...] = xf

    z = _dot32(xf, wcat_ref[...])
    zg = z[:, 0:G]
    ze = z[:, G:G + E]

    zg = zg - jnp.max(zg, axis=1, keepdims=True)
    eg = jnp.exp(zg)
    gp = eg / jnp.sum(eg, axis=1, keepdims=True)

    grow = lax.broadcasted_iota(jnp.int32, (G, E), 0)
    gcol = lax.broadcasted_iota(jnp.int32, (G, E), 1)
    expand = (gcol // EPG == grow).astype(_F32)
    gpf = _dot32(gp, expand)

    ze = ze - jnp.max(ze, axis=1, keepdims=True)
    ee = jnp.exp(ze)
    mr = lax.broadcasted_iota(jnp.int32, (E, E), 0)
    mc = lax.broadcasted_iota(jnp.int32, (E, E), 1)
    gmask = (mr // EPG == mc // EPG).astype(_F32)
    seg = _dot32(ee, gmask)
    probs = gpf * ee / seg

    lane = lax.broadcasted_iota(jnp.int32, (rb, E), 1)
    m1 = jnp.max(probs, axis=1, keepdims=True)
    idx1 = jnp.min(jnp.where(probs == m1, lane, E), axis=1, keepdims=True)
    p2 = jnp.where(lane == idx1, -1.0, probs)
    m2 = jnp.max(p2, axis=1, keepdims=True)
    idx2 = jnp.min(jnp.where(p2 == m2, lane, E), axis=1, keepdims=True)
    denom = m1 + m2 + 1e-9
    g1 = m1 / denom
    g2 = m2 / denom

    row = lax.broadcasted_iota(jnp.int32, (rb, 1), 0)
    valid = (blk * rb + row) < S1
    bf = jnp.bfloat16
    oh1 = ((lane == idx1) & valid).astype(bf)
    oh2 = ((lane == idx2) & valid).astype(bf)
    oh = oh1 + oh2

    tr = lax.broadcasted_iota(jnp.int32, (rb, rb), 0)
    tc = lax.broadcasted_iota(jnp.int32, (rb, rb), 1)
    ltri = (tc < tr).astype(bf)
    carry = cnt_ref[0:1, 0:E]
    excl = _dot(ltri, oh) + carry
    cnt_ref[0:1, 0:E] = carry + jnp.sum(oh.astype(_F32), axis=0, keepdims=True)

    oh1f = oh1.astype(_F32)
    oh2f = oh2.astype(_F32)
    r1 = jnp.sum(oh1f * excl, axis=1, keepdims=True)
    r2 = jnp.sum(oh2f * excl, axis=1, keepdims=True)
    kept1 = valid & (r1 < CAP)
    kept2 = valid & (r2 < CAP)
    d1_ref[...] = jnp.where(kept1, idx1 * CAP + r1.astype(jnp.int32), TRASH)
    d2_ref[...] = jnp.where(kept2, idx2 * CAP + r2.astype(jnp.int32), TRASH)
    s1_ref[...] = jnp.where(kept1, g1, 0.0)
    s2_ref[...] = jnp.where(kept2, g2, 0.0)


def _router(h, ln_g, ln_b, wcat, nb=3, rb=768):
    return pl.pallas_call(
        functools.partial(_router_kernel, rb),
        grid=(nb,),
        in_specs=[
            pl.BlockSpec((rb, D), lambda i: (i, 0)),
            pl.BlockSpec((1, D), lambda i: (0, 0)),
            pl.BlockSpec((1, D), lambda i: (0, 0)),
            pl.BlockSpec((D, G + E), lambda i: (0, 0)),
        ],
        out_specs=[
            pl.BlockSpec((rb, D), lambda i: (i, 0)),
            pl.BlockSpec((rb, 1), lambda i: (i, 0)),
            pl.BlockSpec((rb, 1), lambda i: (i, 0)),
            pl.BlockSpec((rb, 1), lambda i: (i, 0)),
            pl.BlockSpec((rb, 1), lambda i: (i, 0)),
        ],
        out_shape=[
            jax.ShapeDtypeStruct((nb * rb, D), _F32),
            jax.ShapeDtypeStruct((nb * rb, 1), jnp.int32),
            jax.ShapeDtypeStruct((nb * rb, 1), jnp.int32),
            jax.ShapeDtypeStruct((nb * rb, 1), _F32),
            jax.ShapeDtypeStruct((nb * rb, 1), _F32),
        ],
        scratch_shapes=[pltpu.VMEM((8, 128), _F32)],
    )(h, ln_g.reshape(1, D), ln_b.reshape(1, D), wcat)


def _sc_dispatch(xf, d1, d2):
    mesh = plsc.VectorSubcoreMesh(core_axis_name="c", subcore_axis_name="s",
                                  num_cores=2, num_subcores=16)

    @functools.partial(
        pl.kernel,
        out_type=jax.ShapeDtypeStruct((NBUF, D), _F32),
        mesh=mesh,
        scratch_types=[
            pltpu.VMEM((CHUNK,), jnp.int32),
            pltpu.VMEM((CHUNK,), jnp.int32),
            pltpu.VMEM((CHUNK, D), _F32),
            pltpu.SemaphoreType.DMA,
            pltpu.SemaphoreType.DMA,
            pltpu.SemaphoreType.DMA,
        ],
    )
    def k(xf_hbm, d1_hbm, d2_hbm, buf_hbm, idx1_v, idx2_v, rows_v,
          semr, sem1, sem2):
        wid = lax.axis_index("s") * 2 + lax.axis_index("c")
        base = wid * CHUNK
        cr = pltpu.async_copy(xf_hbm.at[pl.ds(base, CHUNK)], rows_v, semr)
        pltpu.sync_copy(d1_hbm.at[pl.ds(base, CHUNK)], idx1_v)
        pltpu.sync_copy(d2_hbm.at[pl.ds(base, CHUNK)], idx2_v)
        cr.wait()
        c1 = pltpu.async_copy(rows_v, buf_hbm.at[idx1_v], sem1)
        c2 = pltpu.async_copy(rows_v, buf_hbm.at[idx2_v], sem2)
        c1.wait()
        c2.wait()

    return k(xf, d1, d2)


def _sc_gather(y, d1, d2):
    mesh = plsc.VectorSubcoreMesh(core_axis_name="c", subcore_axis_name="s",
                                  num_cores=2, num_subcores=16)

    @functools.partial(
        pl.kernel,
        out_type=(jax.ShapeDtypeStruct((SP, D), _F32),
                  jax.ShapeDtypeStruct((SP, D), _F32)),
        mesh=mesh,
        scratch_types=[
            pltpu.VMEM((CHUNK,), jnp.int32),
            pltpu.VMEM((CHUNK,), jnp.int32),
            pltpu.VMEM((CHUNK, D), _F32),
            pltpu.VMEM((CHUNK, D), _F32),
            pltpu.SemaphoreType.DMA,
            pltpu.SemaphoreType.DMA,
            pltpu.SemaphoreType.DMA,
            pltpu.SemaphoreType.DMA,
        ],
    )
    def k(y_hbm, d1_hbm, d2_hbm, g1_hbm, g2_hbm, idx1_v, idx2_v,
          rows1_v, rows2_v, sem1, sem2, semw1, semw2):
        wid = lax.axis_index("s") * 2 + lax.axis_index("c")
        base = wid * CHUNK
        pltpu.sync_copy(d1_hbm.at[pl.ds(base, CHUNK)], idx1_v)
        pltpu.sync_copy(d2_hbm.at[pl.ds(base, CHUNK)], idx2_v)
        c1 = pltpu.async_copy(y_hbm.at[idx1_v], rows1_v, sem1)
        c2 = pltpu.async_copy(y_hbm.at[idx2_v], rows2_v, sem2)
        c1.wait()
        w1 = pltpu.async_copy(rows1_v, g1_hbm.at[pl.ds(base, CHUNK)], semw1)
        c2.wait()
        w2 = pltpu.async_copy(rows2_v, g2_hbm.at[pl.ds(base, CHUNK)], semw2)
        w1.wait()
        w2.wait()

    return k(y, d1, d2)


def _ffn_kernel(buf_ref, w1_ref, b1_ref, w2_ref, b2_ref, y_ref):
    a = _dotbf(buf_ref[...], w1_ref[0]) + b1_ref[0]
    hgelu = jax.nn.gelu(a)
    y_ref[...] = _dotbf(hgelu, w2_ref[0]) + b2_ref[0]


def _ffn(buf, w1, b1, w2, b2):
    return pl.pallas_call(
        _ffn_kernel,
        grid=(E,),
        in_specs=[
            pl.BlockSpec((CAP, D), lambda e: (e, 0)),
            pl.BlockSpec((1, D, DFF), lambda e: (e, 0, 0)),
            pl.BlockSpec((1, 1, DFF), lambda e: (e, 0, 0)),
            pl.BlockSpec((1, DFF, D), lambda e: (e, 0, 0)),
            pl.BlockSpec((1, 1, D), lambda e: (e, 0, 0)),
        ],
        out_specs=pl.BlockSpec((CAP, D), lambda e: (e, 0)),
        out_shape=jax.ShapeDtypeStruct((NBUF, D), _F32),
    )(buf, w1.astype(jnp.bfloat16), b1.reshape(E, 1, DFF),
      w2.astype(jnp.bfloat16), b2.reshape(E, 1, D))


def _combine_kernel(h_ref, g1_ref, g2_ref, s1_ref, s2_ref, out_ref):
    s1 = s1_ref[...]
    s2 = s2_ref[...]
    t1 = jnp.where(s1 > 0.0, s1 * g1_ref[...], 0.0)
    t2 = jnp.where(s2 > 0.0, s2 * g2_ref[...], 0.0)
    out_ref[...] = h_ref[...] + t1 + t2


def _combine(h, gg1, gg2, s1, s2):
    return pl.pallas_call(
        _combine_kernel,
        grid=(NBA,),
        in_specs=[
            pl.BlockSpec((128, D), lambda i: (i, 0)),
            pl.BlockSpec((128, D), lambda i: (i, 0)),
            pl.BlockSpec((128, D), lambda i: (i, 0)),
            pl.BlockSpec((128, 1), lambda i: (i, 0)),
            pl.BlockSpec((128, 1), lambda i: (i, 0)),
        ],
        out_specs=pl.BlockSpec((128, D), lambda i: (i, 0)),
        out_shape=jax.ShapeDtypeStruct((SP, D), _F32),
    )(h, gg1, gg2, s1, s2)


def _head_moe_kernel(eids_ref, h0_ref, xf_ref, s_ref, w1_ref, b1_ref,
                     w2_ref, b2_ref, lg_ref, lb_ref, hw_ref, hb_ref,
                     out_ref, acc_ref):
    i = pl.program_id(0)

    @pl.when(i == 0)
    def _init():
        acc_ref[...] = h0_ref[...]

    a = jax.nn.gelu(_dotbf(xf_ref[0:1, :], w1_ref[0]) + b1_ref[0])
    y0 = _dotbf(a, w2_ref[0]) + b2_ref[0]
    sv = s_ref[pl.ds(i, 1), :]
    acc_ref[0:1, :] = acc_ref[0:1, :] + sv * y0

    @pl.when(i == TOPK - 1)
    def _fin():
        rep = _ln_rows(acc_ref[0:1, :], lg_ref[...], lb_ref[...])
        out_ref[...] = _dot32(rep, hw_ref[...]) + hb_ref[...]


def _head_moe(eids, h0, xf0, svec, w1, b1, w2, b2, lnf_g, lnf_b,
              head_w, head_b):
    spec = pltpu.PrefetchScalarGridSpec(
        num_scalar_prefetch=1,
        grid=(TOPK,),
        in_specs=[
            pl.BlockSpec((8, D), lambda i, e: (0, 0)),
            pl.BlockSpec((8, D), lambda i, e: (0, 0)),
            pl.BlockSpec((TOPK, 1), lambda i, e: (0, 0)),
            pl.BlockSpec((1, D, DFF), lambda i, e: (e[i], 0, 0)),
            pl.BlockSpec((1, 1, DFF), lambda i, e: (e[i], 0, 0)),
            pl.BlockSpec((1, DFF, D), lambda i, e: (e[i], 0, 0)),
            pl.BlockSpec((1, 1, D), lambda i, e: (e[i], 0, 0)),
            pl.BlockSpec((1, D), lambda i, e: (0, 0)),
            pl.BlockSpec((1, D), lambda i, e: (0, 0)),
            pl.BlockSpec((D, NCLS), lambda i, e: (0, 0)),
            pl.BlockSpec((1, NCLS), lambda i, e: (0, 0)),
        ],
        out_specs=pl.BlockSpec((1, NCLS), lambda i, e: (0, 0)),
        scratch_shapes=[pltpu.VMEM((8, D), _F32)],
    )
    return pl.pallas_call(
        _head_moe_kernel,
        grid_spec=spec,
        out_shape=jax.ShapeDtypeStruct((1, NCLS), _F32),
    )(eids, h0, xf0, svec, w1.astype(jnp.bfloat16), b1.reshape(E, 1, DFF),
      w2.astype(jnp.bfloat16), b2.reshape(E, 1, D), lnf_g.reshape(1, D),
      lnf_b.reshape(1, D), head_w, head_b.reshape(1, NCLS))


def kernel(x, W_in, b_in, cls_token, pos_emb, ln1_g, ln1_b, Wqkv, bqkv, Wo, bo,
           ln2_g, ln2_b, Wg, We, W1, b1, W2, b2, lnf_g, lnf_b, head_W, head_b):
    xp = jnp.pad(x[0], ((1, SP - 1 - x.shape[1]), (0, 0)))
    table = jnp.concatenate([
        cls_token[0] + pos_emb[0:1],
        pos_emb[1:S1] + b_in[None, :],
        jnp.zeros((SP - S1, D), _F32),
    ], axis=0)

    h = _embed(xp, W_in, table)

    qkv = _qkv(h, ln1_g[0], ln1_b[0], Wqkv[0], bqkv[0])
    o = _attention(qkv)
    h = _proj_residual(o, h, Wo[0], bo[0])
    wcat = jnp.concatenate(
        [Wg[0], We[0].transpose(1, 0, 2).reshape(D, E)], axis=1)
    xf, d1, d2, s1, s2 = _router(h, ln2_g[0], ln2_b[0], wcat)
    d1f = d1.reshape(SP)
    d2f = d2.reshape(SP)
    buf = _sc_dispatch(xf, d1f, d2f)
    y = _ffn(buf, W1[0], b1[0], W2[0], b2[0])
    gg1, gg2 = _sc_gather(y, d1f, d2f)
    h = _combine(h, gg1, gg2, s1, s2)

    qkv = _qkv(h, ln1_g[1], ln1_b[1], Wqkv[1], bqkv[1])
    o0 = _attention(qkv, nqb=1, qb=128, out_rows=128)
    h0 = _proj_residual(o0, h, Wo[1], bo[1], nb=1, out_rows=128)
    wcat = jnp.concatenate(
        [Wg[1], We[1].transpose(1, 0, 2).reshape(D, E)], axis=1)
    xf0, d1, d2, s1, s2 = _router(h0, ln2_g[1], ln2_b[1], wcat, nb=1, rb=128)
    eids = jnp.stack([d1[0, 0], d2[0, 0]]).astype(jnp.int32) // CAP
    svec = jnp.stack([s1[0, 0], s2[0, 0]]).reshape(TOPK, 1)
    return _head_moe(eids, h0, xf0, svec, W1[1], b1[1], W2[1], b2[1],
                     lnf_g, lnf_b, head_W, head_b)


# speedup vs baseline: 1.1575x; 1.0528x over previous
import functools

import jax
import jax.numpy as jnp
from jax import lax
from jax.experimental import pallas as pl
from jax.experimental.pallas import tpu as pltpu
from jax.experimental.pallas import tpu_sc as plsc

L = 2
D = 768
H = 12
DH = 64
DFF = 1024
G = 4
EPG = 4
E = 16
TOPK = 2
CAP = 320
NCLS = 10
S1 = 2049
SP = 2304
NB = SP // 128
SA = 2176
NBA = SA // 128
NBUF = E * CAP + 8
TRASH = E * CAP
NW = 32
CHUNK = SP // NW

_F32 = jnp.float32


def _dot(a, b):
    return jnp.dot(a, b, preferred_element_type=_F32)


def _dot32(a, b):
    return jnp.dot(a, b, preferred_element_type=_F32)


def _dotbf(a, b):
    return jnp.dot(a.astype(jnp.bfloat16), b, preferred_element_type=_F32)


def _embed_kernel(xp_ref, w_ref, tab_ref, out_ref):
    out_ref[...] = _dot(xp_ref[...], w_ref[...]) + tab_ref[...]


def _embed(xp, w_in, table):
    return pl.pallas_call(
        _embed_kernel,
        grid=(4,),
        in_specs=[
            pl.BlockSpec((576, D), lambda i: (i, 0)),
            pl.BlockSpec((D, D), lambda i: (0, 0)),
            pl.BlockSpec((576, D), lambda i: (i, 0)),
        ],
        out_specs=pl.BlockSpec((576, D), lambda i: (i, 0)),
        out_shape=jax.ShapeDtypeStruct((SP, D), _F32),
    )(xp.astype(jnp.bfloat16), w_in.astype(jnp.bfloat16), table)


def _ln_rows(x, g, b):
    m = jnp.mean(x, axis=1, keepdims=True)
    c = x - m
    v = jnp.mean(c * c, axis=1, keepdims=True)
    return c * lax.rsqrt(v + 1e-5) * g + b


def _qkv_kernel(h_ref, g_ref, b_ref, w_ref, bias_ref, out_ref):
    t = _ln_rows(h_ref[...], g_ref[...], b_ref[...])
    out_ref[...] = (_dotbf(t, w_ref[...]) + bias_ref[...]).astype(jnp.bfloat16)


def _qkv(h, ln_g, ln_b, wqkv, bqkv):
    return pl.pallas_call(
        _qkv_kernel,
        grid=(4,),
        in_specs=[
            pl.BlockSpec((576, D), lambda i: (i, 0)),
            pl.BlockSpec((1, D), lambda i: (0, 0)),
            pl.BlockSpec((1, D), lambda i: (0, 0)),
            pl.BlockSpec((D, 3 * D), lambda i: (0, 0)),
            pl.BlockSpec((1, 3 * D), lambda i: (0, 0)),
        ],
        out_specs=pl.BlockSpec((576, 3 * D), lambda i: (i, 0)),
        out_shape=jax.ShapeDtypeStruct((SP, 3 * D), jnp.bfloat16),
    )(h, ln_g.reshape(1, D), ln_b.reshape(1, D),
      wqkv.astype(jnp.bfloat16), bqkv.reshape(1, 3 * D))


def _attn_kernel(qb, q_ref, k_ref, v_ref, out_ref):
    col = lax.broadcasted_iota(jnp.int32, (qb, SA), 1)
    for off in (0, DH):
        q = q_ref[:, off:off + DH]
        k = k_ref[:, off:off + DH]
        s = lax.dot_general(q, k, (((1,), (1,)), ((), ())),
                            preferred_element_type=_F32) * (1.0 / (DH ** 0.5))
        s = jnp.where(col < S1, s, -1e30)
        m = jnp.max(s, axis=1, keepdims=True)
        p = jnp.exp(s - m).astype(jnp.bfloat16)
        ones = jnp.ones((SA, 1), jnp.bfloat16)
        r = _dot(p, ones)
        pv = _dot(p, v_ref[:, off:off + DH])
        out_ref[:, off:off + DH] = pv / r


def _attention(qkv, nqb=4, qb=576, out_rows=SP):
    return pl.pallas_call(
        functools.partial(_attn_kernel, qb),
        grid=(H // 2, nqb),
        in_specs=[
            pl.BlockSpec((qb, 2 * DH), lambda h, i: (i, h)),
            pl.BlockSpec((SA, 2 * DH), lambda h, i: (0, H // 2 + h)),
            pl.BlockSpec((SA, 2 * DH), lambda h, i: (0, H + h)),
        ],
        out_specs=pl.BlockSpec((qb, 2 * DH), lambda h, i: (i, h)),
        out_shape=jax.ShapeDtypeStruct((out_rows, D), _F32),
    )(qkv, qkv, qkv)


def _proj_kernel(o_ref, h_ref, w_ref, b_ref, out_ref):
    out_ref[...] = h_ref[...] + _dotbf(o_ref[...], w_ref[...]) + b_ref[...]


def _proj_residual(o, h, wo, bo, nb=4, pb=576, out_rows=SP):
    return pl.pallas_call(
        _proj_kernel,
        grid=(nb,),
        in_specs=[
            pl.BlockSpec((pb, D), lambda i: (i, 0)),
            pl.BlockSpec((pb, D), lambda i: (i, 0)),
            pl.BlockSpec((D, D), lambda i: (0, 0)),
            pl.BlockSpec((1, D), lambda i: (0, 0)),
        ],
        out_specs=pl.BlockSpec((pb, D), lambda i: (i, 0)),
        out_shape=jax.ShapeDtypeStruct((out_rows, D), _F32),
    )(o, h, wo.astype(jnp.bfloat16), bo.reshape(1, D))


def _router_kernel(rb, h_ref, g_ref, b_ref, wcat_ref,
                   xf_ref, d1_ref, d2_ref, s1_ref, s2_ref, cnt_ref):
    blk = pl.program_id(0)

    @pl.when(blk == 0)
    def _init():
        cnt_ref[...] = jnp.zeros_like(cnt_ref)

    xf = _ln_rows(h_ref[...], g_ref[...], b_ref[...])
    xf_ref[...] = xf

    z = _dot32(xf, wcat_ref[...])
    zg = z[:, 0:G]
    ze = z[:, G:G + E]

    zg = zg - jnp.max(zg, axis=1, keepdims=True)
    eg = jnp.exp(zg)
    gp = eg / jnp.sum(eg, axis=1, keepdims=True)

    grow = lax.broadcasted_iota(jnp.int32, (G, E), 0)
    gcol = lax.broadcasted_iota(jnp.int32, (G, E), 1)
    expand = (gcol // EPG == grow).astype(_F32)
    gpf = _dot32(gp, expand)

    ze = ze - jnp.max(ze, axis=1, keepdims=True)
    ee = jnp.exp(ze)
    mr = lax.broadcasted_iota(jnp.int32, (E, E), 0)
    mc = lax.broadcasted_iota(jnp.int32, (E, E), 1)
    gmask = (mr // EPG == mc // EPG).astype(_F32)
    seg = _dot32(ee, gmask)
    probs = gpf * ee / seg

    lane = lax.broadcasted_iota(jnp.int32, (rb, E), 1)
    m1 = jnp.max(probs, axis=1, keepdims=True)
    idx1 = jnp.min(jnp.where(probs == m1, lane, E), axis=1, keepdims=True)
    p2 = jnp.where(lane == idx1, -1.0, probs)
    m2 = jnp.max(p2, axis=1, keepdims=True)
    idx2 = jnp.min(jnp.where(p2 == m2, lane, E), axis=1, keepdims=True)
    denom = m1 + m2 + 1e-9
    g1 = m1 / denom
    g2 = m2 / denom

    row = lax.broadcasted_iota(jnp.int32, (rb, 1), 0)
    valid = (blk * rb + row) < S1
    bf = jnp.bfloat16
    oh1 = ((lane == idx1) & valid).astype(bf)
    oh2 = ((lane == idx2) & valid).astype(bf)
    oh = oh1 + oh2

    tr = lax.broadcasted_iota(jnp.int32, (rb, rb), 0)
    tc = lax.broadcasted_iota(jnp.int32, (rb, rb), 1)
    ltri = (tc < tr).astype(bf)
    carry = cnt_ref[0:1, 0:E]
    excl = _dot(ltri, oh) + carry
    cnt_ref[0:1, 0:E] = carry + jnp.sum(oh.astype(_F32), axis=0, keepdims=True)

    oh1f = oh1.astype(_F32)
    oh2f = oh2.astype(_F32)
    r1 = jnp.sum(oh1f * excl, axis=1, keepdims=True)
    r2 = jnp.sum(oh2f * excl, axis=1, keepdims=True)
    kept1 = valid & (r1 < CAP)
    kept2 = valid & (r2 < CAP)
    d1_ref[...] = jnp.where(kept1, idx1 * CAP + r1.astype(jnp.int32), TRASH)
    d2_ref[...] = jnp.where(kept2, idx2 * CAP + r2.astype(jnp.int32), TRASH)
    s1_ref[...] = jnp.where(kept1, g1, 0.0)
    s2_ref[...] = jnp.where(kept2, g2, 0.0)


def _router(h, ln_g, ln_b, wcat, nb=3, rb=768):
    return pl.pallas_call(
        functools.partial(_router_kernel, rb),
        grid=(nb,),
        in_specs=[
            pl.BlockSpec((rb, D), lambda i: (i, 0)),
            pl.BlockSpec((1, D), lambda i: (0, 0)),
            pl.BlockSpec((1, D), lambda i: (0, 0)),
            pl.BlockSpec((D, G + E), lambda i: (0, 0)),
        ],
        out_specs=[
            pl.BlockSpec((rb, D), lambda i: (i, 0)),
            pl.BlockSpec((rb, 1), lambda i: (i, 0)),
            pl.BlockSpec((rb, 1), lambda i: (i, 0)),
            pl.BlockSpec((rb, 1), lambda i: (i, 0)),
            pl.BlockSpec((rb, 1), lambda i: (i, 0)),
        ],
        out_shape=[
            jax.ShapeDtypeStruct((nb * rb, D), _F32),
            jax.ShapeDtypeStruct((nb * rb, 1), jnp.int32),
            jax.ShapeDtypeStruct((nb * rb, 1), jnp.int32),
            jax.ShapeDtypeStruct((nb * rb, 1), _F32),
            jax.ShapeDtypeStruct((nb * rb, 1), _F32),
        ],
        scratch_shapes=[pltpu.VMEM((8, 128), _F32)],
    )(h, ln_g.reshape(1, D), ln_b.reshape(1, D), wcat)


def _sc_dispatch(xf, d1, d2):
    mesh = plsc.VectorSubcoreMesh(core_axis_name="c", subcore_axis_name="s",
                                  num_cores=2, num_subcores=16)

    @functools.partial(
        pl.kernel,
        out_type=jax.ShapeDtypeStruct((NBUF, D), _F32),
        mesh=mesh,
        scratch_types=[
            pltpu.VMEM((CHUNK,), jnp.int32),
            pltpu.VMEM((CHUNK,), jnp.int32),
            pltpu.VMEM((CHUNK, D), _F32),
            pltpu.SemaphoreType.DMA,
            pltpu.SemaphoreType.DMA,
            pltpu.SemaphoreType.DMA,
        ],
    )
    def k(xf_hbm, d1_hbm, d2_hbm, buf_hbm, idx1_v, idx2_v, rows_v,
          semr, sem1, sem2):
        wid = lax.axis_index("s") * 2 + lax.axis_index("c")
        base = wid * CHUNK
        cr = pltpu.async_copy(xf_hbm.at[pl.ds(base, CHUNK)], rows_v, semr)
        pltpu.sync_copy(d1_hbm.at[pl.ds(base, CHUNK)], idx1_v)
        pltpu.sync_copy(d2_hbm.at[pl.ds(base, CHUNK)], idx2_v)
        cr.wait()
        c1 = pltpu.async_copy(rows_v, buf_hbm.at[idx1_v], sem1)
        c2 = pltpu.async_copy(rows_v, buf_hbm.at[idx2_v], sem2)
        c1.wait()
        c2.wait()

    return k(xf, d1, d2)


def _sc_gather(y, d1, d2):
    mesh = plsc.VectorSubcoreMesh(core_axis_name="c", subcore_axis_name="s",
                                  num_cores=2, num_subcores=16)

    @functools.partial(
        pl.kernel,
        out_type=(jax.ShapeDtypeStruct((SP, D), _F32),
                  jax.ShapeDtypeStruct((SP, D), _F32)),
        mesh=mesh,
        scratch_types=[
            pltpu.VMEM((CHUNK,), jnp.int32),
            pltpu.VMEM((CHUNK,), jnp.int32),
            pltpu.VMEM((CHUNK, D), _F32),
            pltpu.VMEM((CHUNK, D), _F32),
            pltpu.SemaphoreType.DMA,
            pltpu.SemaphoreType.DMA,
            pltpu.SemaphoreType.DMA,
            pltpu.SemaphoreType.DMA,
        ],
    )
    def k(y_hbm, d1_hbm, d2_hbm, g1_hbm, g2_hbm, idx1_v, idx2_v,
          rows1_v, rows2_v, sem1, sem2, semw1, semw2):
        wid = lax.axis_index("s") * 2 + lax.axis_index("c")
        base = wid * CHUNK
        pltpu.sync_copy(d1_hbm.at[pl.ds(base, CHUNK)], idx1_v)
        pltpu.sync_copy(d2_hbm.at[pl.ds(base, CHUNK)], idx2_v)
        c1 = pltpu.async_copy(y_hbm.at[idx1_v], rows1_v, sem1)
        c2 = pltpu.async_copy(y_hbm.at[idx2_v], rows2_v, sem2)
        c1.wait()
        w1 = pltpu.async_copy(rows1_v, g1_hbm.at[pl.ds(base, CHUNK)], semw1)
        c2.wait()
        w2 = pltpu.async_copy(rows2_v, g2_hbm.at[pl.ds(base, CHUNK)], semw2)
        w1.wait()
        w2.wait()

    return k(y, d1, d2)


def _ffn_kernel(buf_ref, w1_ref, b1_ref, w2_ref, b2_ref, y_ref):
    a = _dotbf(buf_ref[...], w1_ref[0]) + b1_ref[0]
    hgelu = jax.nn.gelu(a)
    y_ref[...] = _dotbf(hgelu, w2_ref[0]) + b2_ref[0]


def _ffn(buf, w1, b1, w2, b2):
    return pl.pallas_call(
        _ffn_kernel,
        grid=(E,),
        in_specs=[
            pl.BlockSpec((CAP, D), lambda e: (e, 0)),
            pl.BlockSpec((1, D, DFF), lambda e: (e, 0, 0)),
            pl.BlockSpec((1, 1, DFF), lambda e: (e, 0, 0)),
            pl.BlockSpec((1, DFF, D), lambda e: (e, 0, 0)),
            pl.BlockSpec((1, 1, D), lambda e: (e, 0, 0)),
        ],
        out_specs=pl.BlockSpec((CAP, D), lambda e: (e, 0)),
        out_shape=jax.ShapeDtypeStruct((NBUF, D), _F32),
    )(buf, w1.astype(jnp.bfloat16), b1.reshape(E, 1, DFF),
      w2.astype(jnp.bfloat16), b2.reshape(E, 1, D))


def _combine_kernel(h_ref, g1_ref, g2_ref, s1_ref, s2_ref, out_ref):
    s1 = s1_ref[...]
    s2 = s2_ref[...]
    t1 = jnp.where(s1 > 0.0, s1 * g1_ref[...], 0.0)
    t2 = jnp.where(s2 > 0.0, s2 * g2_ref[...], 0.0)
    out_ref[...] = h_ref[...] + t1 + t2


def _combine(h, gg1, gg2, s1, s2):
    return pl.pallas_call(
        _combine_kernel,
        grid=(4,),
        in_specs=[
            pl.BlockSpec((576, D), lambda i: (i, 0)),
            pl.BlockSpec((576, D), lambda i: (i, 0)),
            pl.BlockSpec((576, D), lambda i: (i, 0)),
            pl.BlockSpec((576, 1), lambda i: (i, 0)),
            pl.BlockSpec((576, 1), lambda i: (i, 0)),
        ],
        out_specs=pl.BlockSpec((576, D), lambda i: (i, 0)),
        out_shape=jax.ShapeDtypeStruct((SP, D), _F32),
    )(h, gg1, gg2, s1, s2)


def _head_moe_kernel(eids_ref, h0_ref, xf_ref, s_ref, w1_ref, b1_ref,
                     w2_ref, b2_ref, lg_ref, lb_ref, hw_ref, hb_ref,
                     out_ref, acc_ref):
    i = pl.program_id(0)

    @pl.when(i == 0)
    def _init():
        acc_ref[...] = h0_ref[...]

    a = jax.nn.gelu(_dotbf(xf_ref[0:1, :], w1_ref[0]) + b1_ref[0])
    y0 = _dotbf(a, w2_ref[0]) + b2_ref[0]
    sv = s_ref[pl.ds(i, 1), :]
    acc_ref[0:1, :] = acc_ref[0:1, :] + sv * y0

    @pl.when(i == TOPK - 1)
    def _fin():
        rep = _ln_rows(acc_ref[0:1, :], lg_ref[...], lb_ref[...])
        out_ref[...] = _dot32(rep, hw_ref[...]) + hb_ref[...]


def _head_moe(eids, h0, xf0, svec, w1, b1, w2, b2, lnf_g, lnf_b,
              head_w, head_b):
    spec = pltpu.PrefetchScalarGridSpec(
        num_scalar_prefetch=1,
        grid=(TOPK,),
        in_specs=[
            pl.BlockSpec((8, D), lambda i, e: (0, 0)),
            pl.BlockSpec((8, D), lambda i, e: (0, 0)),
            pl.BlockSpec((TOPK, 1), lambda i, e: (0, 0)),
            pl.BlockSpec((1, D, DFF), lambda i, e: (e[i], 0, 0)),
            pl.BlockSpec((1, 1, DFF), lambda i, e: (e[i], 0, 0)),
            pl.BlockSpec((1, DFF, D), lambda i, e: (e[i], 0, 0)),
            pl.BlockSpec((1, 1, D), lambda i, e: (e[i], 0, 0)),
            pl.BlockSpec((1, D), lambda i, e: (0, 0)),
            pl.BlockSpec((1, D), lambda i, e: (0, 0)),
            pl.BlockSpec((D, NCLS), lambda i, e: (0, 0)),
            pl.BlockSpec((1, NCLS), lambda i, e: (0, 0)),
        ],
        out_specs=pl.BlockSpec((1, NCLS), lambda i, e: (0, 0)),
        scratch_shapes=[pltpu.VMEM((8, D), _F32)],
    )
    return pl.pallas_call(
        _head_moe_kernel,
        grid_spec=spec,
        out_shape=jax.ShapeDtypeStruct((1, NCLS), _F32),
    )(eids, h0, xf0, svec, w1.astype(jnp.bfloat16), b1.reshape(E, 1, DFF),
      w2.astype(jnp.bfloat16), b2.reshape(E, 1, D), lnf_g.reshape(1, D),
      lnf_b.reshape(1, D), head_w, head_b.reshape(1, NCLS))


def kernel(x, W_in, b_in, cls_token, pos_emb, ln1_g, ln1_b, Wqkv, bqkv, Wo, bo,
           ln2_g, ln2_b, Wg, We, W1, b1, W2, b2, lnf_g, lnf_b, head_W, head_b):
    xp = jnp.pad(x[0], ((1, SP - 1 - x.shape[1]), (0, 0)))
    table = jnp.concatenate([
        cls_token[0] + pos_emb[0:1],
        pos_emb[1:S1] + b_in[None, :],
        jnp.zeros((SP - S1, D), _F32),
    ], axis=0)

    h = _embed(xp, W_in, table)

    qkv = _qkv(h, ln1_g[0], ln1_b[0], Wqkv[0], bqkv[0])
    o = _attention(qkv)
    h = _proj_residual(o, h, Wo[0], bo[0])
    wcat = jnp.concatenate(
        [Wg[0], We[0].transpose(1, 0, 2).reshape(D, E)], axis=1)
    xf, d1, d2, s1, s2 = _router(h, ln2_g[0], ln2_b[0], wcat)
    d1f = d1.reshape(SP)
    d2f = d2.reshape(SP)
    buf = _sc_dispatch(xf, d1f, d2f)
    y = _ffn(buf, W1[0], b1[0], W2[0], b2[0])
    gg1, gg2 = _sc_gather(y, d1f, d2f)
    h = _combine(h, gg1, gg2, s1, s2)

    qkv = _qkv(h, ln1_g[1], ln1_b[1], Wqkv[1], bqkv[1])
    o0 = _attention(qkv, nqb=1, qb=128, out_rows=128)
    h0 = _proj_residual(o0, h, Wo[1], bo[1], nb=1, pb=128, out_rows=128)
    wcat = jnp.concatenate(
        [Wg[1], We[1].transpose(1, 0, 2).reshape(D, E)], axis=1)
    xf0, d1, d2, s1, s2 = _router(h0, ln2_g[1], ln2_b[1], wcat, nb=1, rb=128)
    eids = jnp.stack([d1[0, 0], d2[0, 0]]).astype(jnp.int32) // CAP
    svec = jnp.stack([s1[0, 0], s2[0, 0]]).reshape(TOPK, 1)
    return _head_moe(eids, h0, xf0, svec, W1[1], b1[1], W2[1], b2[1],
                     lnf_g, lnf_b, head_W, head_b)


# speedup vs baseline: 1.2011x; 1.0377x over previous
import functools

import jax
import jax.numpy as jnp
from jax import lax
from jax.experimental import pallas as pl
from jax.experimental.pallas import tpu as pltpu
from jax.experimental.pallas import tpu_sc as plsc

L = 2
D = 768
H = 12
DH = 64
DFF = 1024
G = 4
EPG = 4
E = 16
TOPK = 2
CAP = 320
NCLS = 10
S1 = 2049
SP = 2304
NB = SP // 128
SA = 2176
NBA = SA // 128
NBUF = E * CAP + 8
TRASH = E * CAP
NW = 32
CHUNK = SP // NW

_F32 = jnp.float32


def _dot(a, b):
    return jnp.dot(a, b, preferred_element_type=_F32)


def _dot32(a, b):
    return jnp.dot(a, b, preferred_element_type=_F32)


def _dotbf(a, b):
    return jnp.dot(a.astype(jnp.bfloat16), b, preferred_element_type=_F32)


def _embed_qkv_kernel(xp_ref, w_ref, tab_ref, g_ref, b_ref, wq_ref, bias_ref,
                      h_ref, qkv_ref):
    hh = _dot(xp_ref[...], w_ref[...]) + tab_ref[...]
    h_ref[...] = hh
    t = _ln_rows(hh, g_ref[...], b_ref[...])
    qkv_ref[...] = (_dotbf(t, wq_ref[...]) + bias_ref[...]).astype(jnp.bfloat16)


def _embed_qkv(xp, w_in, table, ln_g, ln_b, wqkv, bqkv):
    return pl.pallas_call(
        _embed_qkv_kernel,
        grid=(4,),
        in_specs=[
            pl.BlockSpec((576, D), lambda i: (i, 0)),
            pl.BlockSpec((D, D), lambda i: (0, 0)),
            pl.BlockSpec((576, D), lambda i: (i, 0)),
            pl.BlockSpec((1, D), lambda i: (0, 0)),
            pl.BlockSpec((1, D), lambda i: (0, 0)),
            pl.BlockSpec((D, 3 * D), lambda i: (0, 0)),
            pl.BlockSpec((1, 3 * D), lambda i: (0, 0)),
        ],
        out_specs=[
            pl.BlockSpec((576, D), lambda i: (i, 0)),
            pl.BlockSpec((576, 3 * D), lambda i: (i, 0)),
        ],
        out_shape=[
            jax.ShapeDtypeStruct((SP, D), _F32),
            jax.ShapeDtypeStruct((SP, 3 * D), jnp.bfloat16),
        ],
    )(xp.astype(jnp.bfloat16), w_in.astype(jnp.bfloat16), table,
      ln_g.reshape(1, D), ln_b.reshape(1, D),
      wqkv.astype(jnp.bfloat16), bqkv.reshape(1, 3 * D))


def _ln_rows(x, g, b):
    m = jnp.mean(x, axis=1, keepdims=True)
    c = x - m
    v = jnp.mean(c * c, axis=1, keepdims=True)
    return c * lax.rsqrt(v + 1e-5) * g + b


def _qkv_kernel(h_ref, g_ref, b_ref, w_ref, bias_ref, out_ref):
    t = _ln_rows(h_ref[...], g_ref[...], b_ref[...])
    out_ref[...] = (_dotbf(t, w_ref[...]) + bias_ref[...]).astype(jnp.bfloat16)


def _qkv(h, ln_g, ln_b, wqkv, bqkv):
    return pl.pallas_call(
        _qkv_kernel,
        grid=(4,),
        in_specs=[
            pl.BlockSpec((576, D), lambda i: (i, 0)),
            pl.BlockSpec((1, D), lambda i: (0, 0)),
            pl.BlockSpec((1, D), lambda i: (0, 0)),
            pl.BlockSpec((D, 3 * D), lambda i: (0, 0)),
            pl.BlockSpec((1, 3 * D), lambda i: (0, 0)),
        ],
        out_specs=pl.BlockSpec((576, 3 * D), lambda i: (i, 0)),
        out_shape=jax.ShapeDtypeStruct((SP, 3 * D), jnp.bfloat16),
    )(h, ln_g.reshape(1, D), ln_b.reshape(1, D),
      wqkv.astype(jnp.bfloat16), bqkv.reshape(1, 3 * D))


def _attn_kernel(qb, q_ref, k_ref, v_ref, out_ref):
    col = lax.broadcasted_iota(jnp.int32, (qb, SA), 1)
    for off in (0, DH):
        q = q_ref[:, off:off + DH]
        k = k_ref[:, off:off + DH]
        s = lax.dot_general(q, k, (((1,), (1,)), ((), ())),
                            preferred_element_type=_F32) * (1.0 / (DH ** 0.5))
        s = jnp.where(col < S1, s, -1e30)
        m = jnp.max(s, axis=1, keepdims=True)
        p = jnp.exp(s - m).astype(jnp.bfloat16)
        ones = jnp.ones((SA, 1), jnp.bfloat16)
        r = _dot(p, ones)
        pv = _dot(p, v_ref[:, off:off + DH])
        out_ref[:, off:off + DH] = pv / r


def _attention(qkv, nqb=4, qb=576, out_rows=SP):
    return pl.pallas_call(
        functools.partial(_attn_kernel, qb),
        grid=(H // 2, nqb),
        in_specs=[
            pl.BlockSpec((qb, 2 * DH), lambda h, i: (i, h)),
            pl.BlockSpec((SA, 2 * DH), lambda h, i: (0, H // 2 + h)),
            pl.BlockSpec((SA, 2 * DH), lambda h, i: (0, H + h)),
        ],
        out_specs=pl.BlockSpec((qb, 2 * DH), lambda h, i: (i, h)),
        out_shape=jax.ShapeDtypeStruct((out_rows, D), _F32),
    )(qkv, qkv, qkv)


def _proj_kernel(o_ref, h_ref, w_ref, b_ref, out_ref):
    out_ref[...] = h_ref[...] + _dotbf(o_ref[...], w_ref[...]) + b_ref[...]


def _proj_residual(o, h, wo, bo, nb=4, pb=576, out_rows=SP):
    return pl.pallas_call(
        _proj_kernel,
        grid=(nb,),
        in_specs=[
            pl.BlockSpec((pb, D), lambda i: (i, 0)),
            pl.BlockSpec((pb, D), lambda i: (i, 0)),
            pl.BlockSpec((D, D), lambda i: (0, 0)),
            pl.BlockSpec((1, D), lambda i: (0, 0)),
        ],
        out_specs=pl.BlockSpec((pb, D), lambda i: (i, 0)),
        out_shape=jax.ShapeDtypeStruct((out_rows, D), _F32),
    )(o, h, wo.astype(jnp.bfloat16), bo.reshape(1, D))


def _router_kernel(rb, o_ref, h_ref, wo_ref, bo_ref, g_ref, b_ref, wcat_ref,
                   hn_ref, xf_ref, d1_ref, d2_ref, s1_ref, s2_ref, cnt_ref):
    blk = pl.program_id(0)

    @pl.when(blk == 0)
    def _init():
        cnt_ref[...] = jnp.zeros_like(cnt_ref)

    hn = h_ref[...] + _dotbf(o_ref[...], wo_ref[...]) + bo_ref[...]
    hn_ref[...] = hn
    xf = _ln_rows(hn, g_ref[...], b_ref[...])
    xf_ref[...] = xf

    z = _dot32(xf, wcat_ref[...])
    zg = z[:, 0:G]
    ze = z[:, G:G + E]

    zg = zg - jnp.max(zg, axis=1, keepdims=True)
    eg = jnp.exp(zg)
    gp = eg / jnp.sum(eg, axis=1, keepdims=True)

    grow = lax.broadcasted_iota(jnp.int32, (G, E), 0)
    gcol = lax.broadcasted_iota(jnp.int32, (G, E), 1)
    expand = (gcol // EPG == grow).astype(_F32)
    gpf = _dot32(gp, expand)

    ze = ze - jnp.max(ze, axis=1, keepdims=True)
    ee = jnp.exp(ze)
    mr = lax.broadcasted_iota(jnp.int32, (E, E), 0)
    mc = lax.broadcasted_iota(jnp.int32, (E, E), 1)
    gmask = (mr // EPG == mc // EPG).astype(_F32)
    seg = _dot32(ee, gmask)
    probs = gpf * ee / seg

    lane = lax.broadcasted_iota(jnp.int32, (rb, E), 1)
    m1 = jnp.max(probs, axis=1, keepdims=True)
    idx1 = jnp.min(jnp.where(probs == m1, lane, E), axis=1, keepdims=True)
    p2 = jnp.where(lane == idx1, -1.0, probs)
    m2 = jnp.max(p2, axis=1, keepdims=True)
    idx2 = jnp.min(jnp.where(p2 == m2, lane, E), axis=1, keepdims=True)
    denom = m1 + m2 + 1e-9
    g1 = m1 / denom
    g2 = m2 / denom

    row = lax.broadcasted_iota(jnp.int32, (rb, 1), 0)
    valid = (blk * rb + row) < S1
    bf = jnp.bfloat16
    oh1 = ((lane == idx1) & valid).astype(bf)
    oh2 = ((lane == idx2) & valid).astype(bf)
    oh = oh1 + oh2

    tr = lax.broadcasted_iota(jnp.int32, (rb, rb), 0)
    tc = lax.broadcasted_iota(jnp.int32, (rb, rb), 1)
    ltri = (tc < tr).astype(bf)
    carry = cnt_ref[0:1, 0:E]
    excl = _dot(ltri, oh) + carry
    cnt_ref[0:1, 0:E] = carry + jnp.sum(oh.astype(_F32), axis=0, keepdims=True)

    oh1f = oh1.astype(_F32)
    oh2f = oh2.astype(_F32)
    r1 = jnp.sum(oh1f * excl, axis=1, keepdims=True)
    r2 = jnp.sum(oh2f * excl, axis=1, keepdims=True)
    kept1 = valid & (r1 < CAP)
    kept2 = valid & (r2 < CAP)
    d1_ref[...] = jnp.where(kept1, idx1 * CAP + r1.astype(jnp.int32), TRASH)
    d2_ref[...] = jnp.where(kept2, idx2 * CAP + r2.astype(jnp.int32), TRASH)
    s1_ref[...] = jnp.where(kept1, g1, 0.0)
    s2_ref[...] = jnp.where(kept2, g2, 0.0)


def _router(o, h, wo, bo, ln_g, ln_b, wcat, nb=3, rb=768):
    return pl.pallas_call(
        functools.partial(_router_kernel, rb),
        grid=(nb,),
        in_specs=[
            pl.BlockSpec((rb, D), lambda i: (i, 0)),
            pl.BlockSpec((rb, D), lambda i: (i, 0)),
            pl.BlockSpec((D, D), lambda i: (0, 0)),
            pl.BlockSpec((1, D), lambda i: (0, 0)),
            pl.BlockSpec((1, D), lambda i: (0, 0)),
            pl.BlockSpec((1, D), lambda i: (0, 0)),
            pl.BlockSpec((D, G + E), lambda i: (0, 0)),
        ],
        out_specs=[
            pl.BlockSpec((rb, D), lambda i: (i, 0)),
            pl.BlockSpec((rb, D), lambda i: (i, 0)),
            pl.BlockSpec((rb, 1), lambda i: (i, 0)),
            pl.BlockSpec((rb, 1), lambda i: (i, 0)),
            pl.BlockSpec((rb, 1), lambda i: (i, 0)),
            pl.BlockSpec((rb, 1), lambda i: (i, 0)),
        ],
        out_shape=[
            jax.ShapeDtypeStruct((nb * rb, D), _F32),
            jax.ShapeDtypeStruct((nb * rb, D), _F32),
            jax.ShapeDtypeStruct((nb * rb, 1), jnp.int32),
            jax.ShapeDtypeStruct((nb * rb, 1), jnp.int32),
            jax.ShapeDtypeStruct((nb * rb, 1), _F32),
            jax.ShapeDtypeStruct((nb * rb, 1), _F32),
        ],
        scratch_shapes=[pltpu.VMEM((8, 128), _F32)],
    )(o, h, wo.astype(jnp.bfloat16), bo.reshape(1, D),
      ln_g.reshape(1, D), ln_b.reshape(1, D), wcat)


def _sc_dispatch(xf, d1, d2):
    mesh = plsc.VectorSubcoreMesh(core_axis_name="c", subcore_axis_name="s",
                                  num_cores=2, num_subcores=16)

    @functools.partial(
        pl.kernel,
        out_type=jax.ShapeDtypeStruct((NBUF, D), _F32),
        mesh=mesh,
        scratch_types=[
            pltpu.VMEM((CHUNK,), jnp.int32),
            pltpu.VMEM((CHUNK,), jnp.int32),
            pltpu.VMEM((CHUNK, D), _F32),
            pltpu.SemaphoreType.DMA,
            pltpu.SemaphoreType.DMA,
            pltpu.SemaphoreType.DMA,
        ],
    )
    def k(xf_hbm, d1_hbm, d2_hbm, buf_hbm, idx1_v, idx2_v, rows_v,
          semr, sem1, sem2):
        wid = lax.axis_index("s") * 2 + lax.axis_index("c")
        base = wid * CHUNK
        cr = pltpu.async_copy(xf_hbm.at[pl.ds(base, CHUNK)], rows_v, semr)
        pltpu.sync_copy(d1_hbm.at[pl.ds(base, CHUNK)], idx1_v)
        pltpu.sync_copy(d2_hbm.at[pl.ds(base, CHUNK)], idx2_v)
        cr.wait()
        c1 = pltpu.async_copy(rows_v, buf_hbm.at[idx1_v], sem1)
        c2 = pltpu.async_copy(rows_v, buf_hbm.at[idx2_v], sem2)
        c1.wait()
        c2.wait()

    return k(xf, d1, d2)


def _sc_gather(y, d1, d2):
    mesh = plsc.VectorSubcoreMesh(core_axis_name="c", subcore_axis_name="s",
                                  num_cores=2, num_subcores=16)

    @functools.partial(
        pl.kernel,
        out_type=(jax.ShapeDtypeStruct((SP, D), _F32),
                  jax.ShapeDtypeStruct((SP, D), _F32)),
        mesh=mesh,
        scratch_types=[
            pltpu.VMEM((CHUNK,), jnp.int32),
            pltpu.VMEM((CHUNK,), jnp.int32),
            pltpu.VMEM((CHUNK, D), _F32),
            pltpu.VMEM((CHUNK, D), _F32),
            pltpu.SemaphoreType.DMA,
            pltpu.SemaphoreType.DMA,
            pltpu.SemaphoreType.DMA,
            pltpu.SemaphoreType.DMA,
        ],
    )
    def k(y_hbm, d1_hbm, d2_hbm, g1_hbm, g2_hbm, idx1_v, idx2_v,
          rows1_v, rows2_v, sem1, sem2, semw1, semw2):
        wid = lax.axis_index("s") * 2 + lax.axis_index("c")
        base = wid * CHUNK
        pltpu.sync_copy(d1_hbm.at[pl.ds(base, CHUNK)], idx1_v)
        pltpu.sync_copy(d2_hbm.at[pl.ds(base, CHUNK)], idx2_v)
        c1 = pltpu.async_copy(y_hbm.at[idx1_v], rows1_v, sem1)
        c2 = pltpu.async_copy(y_hbm.at[idx2_v], rows2_v, sem2)
        c1.wait()
        w1 = pltpu.async_copy(rows1_v, g1_hbm.at[pl.ds(base, CHUNK)], semw1)
        c2.wait()
        w2 = pltpu.async_copy(rows2_v, g2_hbm.at[pl.ds(base, CHUNK)], semw2)
        w1.wait()
        w2.wait()

    return k(y, d1, d2)


def _ffn_kernel(buf_ref, w1_ref, b1_ref, w2_ref, b2_ref, y_ref):
    a = _dotbf(buf_ref[...], w1_ref[0]) + b1_ref[0]
    hgelu = jax.nn.gelu(a)
    y_ref[...] = _dotbf(hgelu, w2_ref[0]) + b2_ref[0]


def _ffn(buf, w1, b1, w2, b2):
    return pl.pallas_call(
        _ffn_kernel,
        grid=(E,),
        in_specs=[
            pl.BlockSpec((CAP, D), lambda e: (e, 0)),
            pl.BlockSpec((1, D, DFF), lambda e: (e, 0, 0)),
            pl.BlockSpec((1, 1, DFF), lambda e: (e, 0, 0)),
            pl.BlockSpec((1, DFF, D), lambda e: (e, 0, 0)),
            pl.BlockSpec((1, 1, D), lambda e: (e, 0, 0)),
        ],
        out_specs=pl.BlockSpec((CAP, D), lambda e: (e, 0)),
        out_shape=jax.ShapeDtypeStruct((NBUF, D), _F32),
    )(buf, w1.astype(jnp.bfloat16), b1.reshape(E, 1, DFF),
      w2.astype(jnp.bfloat16), b2.reshape(E, 1, D))


def _combine_qkv_kernel(h_ref, g1_ref, g2_ref, s1_ref, s2_ref,
                        lg_ref, lb_ref, wq_ref, bias_ref, hc_ref, qkv_ref):
    s1 = s1_ref[...]
    s2 = s2_ref[...]
    t1 = jnp.where(s1 > 0.0, s1 * g1_ref[...], 0.0)
    t2 = jnp.where(s2 > 0.0, s2 * g2_ref[...], 0.0)
    hc = h_ref[...] + t1 + t2
    hc_ref[...] = hc
    t = _ln_rows(hc, lg_ref[...], lb_ref[...])
    qkv_ref[...] = (_dotbf(t, wq_ref[...]) + bias_ref[...]).astype(jnp.bfloat16)


def _combine_qkv(h, gg1, gg2, s1, s2, ln_g, ln_b, wqkv, bqkv):
    return pl.pallas_call(
        _combine_qkv_kernel,
        grid=(4,),
        in_specs=[
            pl.BlockSpec((576, D), lambda i: (i, 0)),
            pl.BlockSpec((576, D), lambda i: (i, 0)),
            pl.BlockSpec((576, D), lambda i: (i, 0)),
            pl.BlockSpec((576, 1), lambda i: (i, 0)),
            pl.BlockSpec((576, 1), lambda i: (i, 0)),
            pl.BlockSpec((1, D), lambda i: (0, 0)),
            pl.BlockSpec((1, D), lambda i: (0, 0)),
            pl.BlockSpec((D, 3 * D), lambda i: (0, 0)),
            pl.BlockSpec((1, 3 * D), lambda i: (0, 0)),
        ],
        out_specs=[
            pl.BlockSpec((576, D), lambda i: (i, 0)),
            pl.BlockSpec((576, 3 * D), lambda i: (i, 0)),
        ],
        out_shape=[
            jax.ShapeDtypeStruct((SP, D), _F32),
            jax.ShapeDtypeStruct((SP, 3 * D), jnp.bfloat16),
        ],
    )(h, gg1, gg2, s1, s2, ln_g.reshape(1, D), ln_b.reshape(1, D),
      wqkv.astype(jnp.bfloat16), bqkv.reshape(1, 3 * D))


def _head_moe_kernel(eids_ref, h0_ref, xf_ref, s_ref, w1_ref, b1_ref,
                     w2_ref, b2_ref, lg_ref, lb_ref, hw_ref, hb_ref,
                     out_ref, acc_ref):
    i = pl.program_id(0)

    @pl.when(i == 0)
    def _init():
        acc_ref[...] = h0_ref[...]

    a = jax.nn.gelu(_dotbf(xf_ref[0:1, :], w1_ref[0]) + b1_ref[0])
    y0 = _dotbf(a, w2_ref[0]) + b2_ref[0]
    sv = s_ref[pl.ds(i, 1), :]
    acc_ref[0:1, :] = acc_ref[0:1, :] + sv * y0

    @pl.when(i == TOPK - 1)
    def _fin():
        rep = _ln_rows(acc_ref[0:1, :], lg_ref[...], lb_ref[...])
        out_ref[...] = _dot32(rep, hw_ref[...]) + hb_ref[...]


def _head_moe(eids, h0, xf0, svec, w1, b1, w2, b2, lnf_g, lnf_b,
              head_w, head_b):
    spec = pltpu.PrefetchScalarGridSpec(
        num_scalar_prefetch=1,
        grid=(TOPK,),
        in_specs=[
            pl.BlockSpec((8, D), lambda i, e: (0, 0)),
            pl.BlockSpec((8, D), lambda i, e: (0, 0)),
            pl.BlockSpec((TOPK, 1), lambda i, e: (0, 0)),
            pl.BlockSpec((1, D, DFF), lambda i, e: (e[i], 0, 0)),
            pl.BlockSpec((1, 1, DFF), lambda i, e: (e[i], 0, 0)),
            pl.BlockSpec((1, DFF, D), lambda i, e: (e[i], 0, 0)),
            pl.BlockSpec((1, 1, D), lambda i, e: (e[i], 0, 0)),
            pl.BlockSpec((1, D), lambda i, e: (0, 0)),
            pl.BlockSpec((1, D), lambda i, e: (0, 0)),
            pl.BlockSpec((D, NCLS), lambda i, e: (0, 0)),
            pl.BlockSpec((1, NCLS), lambda i, e: (0, 0)),
        ],
        out_specs=pl.BlockSpec((1, NCLS), lambda i, e: (0, 0)),
        scratch_shapes=[pltpu.VMEM((8, D), _F32)],
    )
    return pl.pallas_call(
        _head_moe_kernel,
        grid_spec=spec,
        out_shape=jax.ShapeDtypeStruct((1, NCLS), _F32),
    )(eids, h0, xf0, svec, w1.astype(jnp.bfloat16), b1.reshape(E, 1, DFF),
      w2.astype(jnp.bfloat16), b2.reshape(E, 1, D), lnf_g.reshape(1, D),
      lnf_b.reshape(1, D), head_w, head_b.reshape(1, NCLS))


def kernel(x, W_in, b_in, cls_token, pos_emb, ln1_g, ln1_b, Wqkv, bqkv, Wo, bo,
           ln2_g, ln2_b, Wg, We, W1, b1, W2, b2, lnf_g, lnf_b, head_W, head_b):
    xp = jnp.pad(x[0], ((1, SP - 1 - x.shape[1]), (0, 0)))
    table = jnp.concatenate([
        cls_token[0] + pos_emb[0:1],
        pos_emb[1:S1] + b_in[None, :],
        jnp.zeros((SP - S1, D), _F32),
    ], axis=0)

    h, qkv = _embed_qkv(xp, W_in, table, ln1_g[0], ln1_b[0], Wqkv[0], bqkv[0])
    o = _attention(qkv)
    wcat = jnp.concatenate(
        [Wg[0], We[0].transpose(1, 0, 2).reshape(D, E)], axis=1)
    h, xf, d1, d2, s1, s2 = _router(o, h, Wo[0], bo[0],
                                    ln2_g[0], ln2_b[0], wcat)
    d1f = d1.reshape(SP)
    d2f = d2.reshape(SP)
    buf = _sc_dispatch(xf, d1f, d2f)
    y = _ffn(buf, W1[0], b1[0], W2[0], b2[0])
    gg1, gg2 = _sc_gather(y, d1f, d2f)

    h, qkv = _combine_qkv(h, gg1, gg2, s1, s2,
                          ln1_g[1], ln1_b[1], Wqkv[1], bqkv[1])
    o0 = _attention(qkv, nqb=1, qb=128, out_rows=128)
    wcat = jnp.concatenate(
        [Wg[1], We[1].transpose(1, 0, 2).reshape(D, E)], axis=1)
    h0, xf0, d1, d2, s1, s2 = _router(o0, h, Wo[1], bo[1],
                                      ln2_g[1], ln2_b[1], wcat, nb=1, rb=128)
    eids = jnp.stack([d1[0, 0], d2[0, 0]]).astype(jnp.int32) // CAP
    svec = jnp.stack([s1[0, 0], s2[0, 0]]).reshape(TOPK, 1)
    return _head_moe(eids, h0, xf0, svec, W1[1], b1[1], W2[1], b2[1],
                     lnf_g, lnf_b, head_W, head_b)


# speedup vs baseline: 1.2203x; 1.0160x over previous
import functools

import jax
import jax.numpy as jnp
from jax import lax
from jax.experimental import pallas as pl
from jax.experimental.pallas import tpu as pltpu
from jax.experimental.pallas import tpu_sc as plsc

L = 2
D = 768
H = 12
DH = 64
DFF = 1024
G = 4
EPG = 4
E = 16
TOPK = 2
CAP = 320
NCLS = 10
S1 = 2049
SP = 2304
NB = SP // 128
SA = 2176
NBA = SA // 128
NBUF = E * CAP + 8
TRASH = E * CAP
NW = 32
CHUNK = SP // NW

_F32 = jnp.float32


def _dot(a, b):
    return jnp.dot(a, b, preferred_element_type=_F32)


def _dot32(a, b):
    return jnp.dot(a, b, preferred_element_type=_F32)


def _dotbf(a, b):
    return jnp.dot(a.astype(jnp.bfloat16), b, preferred_element_type=_F32)


def _embed_qkv_kernel(xp_ref, w_ref, tab_ref, g_ref, b_ref, wq_ref, bias_ref,
                      h_ref, qkv_ref):
    hh = _dot(xp_ref[...], w_ref[...]) + tab_ref[...]
    h_ref[...] = hh
    t = _ln_rows(hh, g_ref[...], b_ref[...])
    qkv_ref[...] = (_dotbf(t, wq_ref[...]) + bias_ref[...]).astype(jnp.bfloat16)


def _embed_qkv(xp, w_in, table, ln_g, ln_b, wqkv, bqkv):
    return pl.pallas_call(
        _embed_qkv_kernel,
        grid=(4,),
        in_specs=[
            pl.BlockSpec((576, D), lambda i: (i, 0)),
            pl.BlockSpec((D, D), lambda i: (0, 0)),
            pl.BlockSpec((576, D), lambda i: (i, 0)),
            pl.BlockSpec((1, D), lambda i: (0, 0)),
            pl.BlockSpec((1, D), lambda i: (0, 0)),
            pl.BlockSpec((D, 3 * D), lambda i: (0, 0)),
            pl.BlockSpec((1, 3 * D), lambda i: (0, 0)),
        ],
        out_specs=[
            pl.BlockSpec((576, D), lambda i: (i, 0)),
            pl.BlockSpec((576, 3 * D), lambda i: (i, 0)),
        ],
        out_shape=[
            jax.ShapeDtypeStruct((SP, D), _F32),
            jax.ShapeDtypeStruct((SP, 3 * D), jnp.bfloat16),
        ],
    )(xp.astype(jnp.bfloat16), w_in.astype(jnp.bfloat16), table,
      ln_g.reshape(1, D), ln_b.reshape(1, D),
      wqkv.astype(jnp.bfloat16), bqkv.reshape(1, 3 * D))


def _ln_rows(x, g, b):
    m = jnp.mean(x, axis=1, keepdims=True)
    c = x - m
    v = jnp.mean(c * c, axis=1, keepdims=True)
    return c * lax.rsqrt(v + 1e-5) * g + b


def _qkv_kernel(h_ref, g_ref, b_ref, w_ref, bias_ref, out_ref):
    t = _ln_rows(h_ref[...], g_ref[...], b_ref[...])
    out_ref[...] = (_dotbf(t, w_ref[...]) + bias_ref[...]).astype(jnp.bfloat16)


def _qkv(h, ln_g, ln_b, wqkv, bqkv):
    return pl.pallas_call(
        _qkv_kernel,
        grid=(4,),
        in_specs=[
            pl.BlockSpec((576, D), lambda i: (i, 0)),
            pl.BlockSpec((1, D), lambda i: (0, 0)),
            pl.BlockSpec((1, D), lambda i: (0, 0)),
            pl.BlockSpec((D, 3 * D), lambda i: (0, 0)),
            pl.BlockSpec((1, 3 * D), lambda i: (0, 0)),
        ],
        out_specs=pl.BlockSpec((576, 3 * D), lambda i: (i, 0)),
        out_shape=jax.ShapeDtypeStruct((SP, 3 * D), jnp.bfloat16),
    )(h, ln_g.reshape(1, D), ln_b.reshape(1, D),
      wqkv.astype(jnp.bfloat16), bqkv.reshape(1, 3 * D))


def _attn_kernel(qb, q_ref, k_ref, v_ref, out_ref):
    col = lax.broadcasted_iota(jnp.int32, (qb, SA), 1)
    for off in (0, DH):
        q = q_ref[:, off:off + DH]
        k = k_ref[:, off:off + DH]
        s = lax.dot_general(q, k, (((1,), (1,)), ((), ())),
                            preferred_element_type=_F32) * (1.0 / (DH ** 0.5))
        s = jnp.where(col < S1, s, -1e30)
        m = jnp.max(s, axis=1, keepdims=True)
        p = jnp.exp(s - m).astype(jnp.bfloat16)
        ones = jnp.ones((SA, 1), jnp.bfloat16)
        r = _dot(p, ones)
        pv = _dot(p, v_ref[:, off:off + DH])
        out_ref[:, off:off + DH] = (pv / r).astype(jnp.bfloat16)


def _attention(qkv, nqb=2, qb=1152, out_rows=SP):
    return pl.pallas_call(
        functools.partial(_attn_kernel, qb),
        grid=(H // 2, nqb),
        in_specs=[
            pl.BlockSpec((qb, 2 * DH), lambda h, i: (i, h)),
            pl.BlockSpec((SA, 2 * DH), lambda h, i: (0, H // 2 + h)),
            pl.BlockSpec((SA, 2 * DH), lambda h, i: (0, H + h)),
        ],
        out_specs=pl.BlockSpec((qb, 2 * DH), lambda h, i: (i, h)),
        out_shape=jax.ShapeDtypeStruct((out_rows, D), jnp.bfloat16),
    )(qkv, qkv, qkv)


def _proj_kernel(o_ref, h_ref, w_ref, b_ref, out_ref):
    out_ref[...] = h_ref[...] + _dotbf(o_ref[...], w_ref[...]) + b_ref[...]


def _proj_residual(o, h, wo, bo, nb=4, pb=576, out_rows=SP):
    return pl.pallas_call(
        _proj_kernel,
        grid=(nb,),
        in_specs=[
            pl.BlockSpec((pb, D), lambda i: (i, 0)),
            pl.BlockSpec((pb, D), lambda i: (i, 0)),
            pl.BlockSpec((D, D), lambda i: (0, 0)),
            pl.BlockSpec((1, D), lambda i: (0, 0)),
        ],
        out_specs=pl.BlockSpec((pb, D), lambda i: (i, 0)),
        out_shape=jax.ShapeDtypeStruct((out_rows, D), _F32),
    )(o, h, wo.astype(jnp.bfloat16), bo.reshape(1, D))


def _router_kernel(rb, o_ref, h_ref, wo_ref, bo_ref, g_ref, b_ref, wcat_ref,
                   hn_ref, xf_ref, d1_ref, d2_ref, s1_ref, s2_ref, cnt_ref):
    blk = pl.program_id(0)

    @pl.when(blk == 0)
    def _init():
        cnt_ref[...] = jnp.zeros_like(cnt_ref)

    hn = h_ref[...] + _dotbf(o_ref[...], wo_ref[...]) + bo_ref[...]
    hn_ref[...] = hn
    xf = _ln_rows(hn, g_ref[...], b_ref[...])
    xf_ref[...] = xf

    z = _dot32(xf, wcat_ref[...])
    zg = z[:, 0:G]
    ze = z[:, G:G + E]

    zg = zg - jnp.max(zg, axis=1, keepdims=True)
    eg = jnp.exp(zg)
    gp = eg / jnp.sum(eg, axis=1, keepdims=True)

    grow = lax.broadcasted_iota(jnp.int32, (G, E), 0)
    gcol = lax.broadcasted_iota(jnp.int32, (G, E), 1)
    expand = (gcol // EPG == grow).astype(_F32)
    gpf = _dot32(gp, expand)

    ze = ze - jnp.max(ze, axis=1, keepdims=True)
    ee = jnp.exp(ze)
    mr = lax.broadcasted_iota(jnp.int32, (E, E), 0)
    mc = lax.broadcasted_iota(jnp.int32, (E, E), 1)
    gmask = (mr // EPG == mc // EPG).astype(_F32)
    seg = _dot32(ee, gmask)
    probs = gpf * ee / seg

    lane = lax.broadcasted_iota(jnp.int32, (rb, E), 1)
    m1 = jnp.max(probs, axis=1, keepdims=True)
    idx1 = jnp.min(jnp.where(probs == m1, lane, E), axis=1, keepdims=True)
    p2 = jnp.where(lane == idx1, -1.0, probs)
    m2 = jnp.max(p2, axis=1, keepdims=True)
    idx2 = jnp.min(jnp.where(p2 == m2, lane, E), axis=1, keepdims=True)
    denom = m1 + m2 + 1e-9
    g1 = m1 / denom
    g2 = m2 / denom

    row = lax.broadcasted_iota(jnp.int32, (rb, 1), 0)
    valid = (blk * rb + row) < S1
    bf = jnp.bfloat16
    oh1 = ((lane == idx1) & valid).astype(bf)
    oh2 = ((lane == idx2) & valid).astype(bf)
    oh = oh1 + oh2

    tr = lax.broadcasted_iota(jnp.int32, (rb, rb), 0)
    tc = lax.broadcasted_iota(jnp.int32, (rb, rb), 1)
    ltri = (tc < tr).astype(bf)
    carry = cnt_ref[0:1, 0:E]
    excl = _dot(ltri, oh) + carry
    cnt_ref[0:1, 0:E] = carry + jnp.sum(oh.astype(_F32), axis=0, keepdims=True)

    oh1f = oh1.astype(_F32)
    oh2f = oh2.astype(_F32)
    r1 = jnp.sum(oh1f * excl, axis=1, keepdims=True)
    r2 = jnp.sum(oh2f * excl, axis=1, keepdims=True)
    kept1 = valid & (r1 < CAP)
    kept2 = valid & (r2 < CAP)
    d1_ref[...] = jnp.where(kept1, idx1 * CAP + r1.astype(jnp.int32), TRASH)
    d2_ref[...] = jnp.where(kept2, idx2 * CAP + r2.astype(jnp.int32), TRASH)
    s1_ref[...] = jnp.where(kept1, g1, 0.0)
    s2_ref[...] = jnp.where(kept2, g2, 0.0)


def _router(o, h, wo, bo, ln_g, ln_b, wcat, nb=3, rb=768):
    return pl.pallas_call(
        functools.partial(_router_kernel, rb),
        grid=(nb,),
        in_specs=[
            pl.BlockSpec((rb, D), lambda i: (i, 0)),
            pl.BlockSpec((rb, D), lambda i: (i, 0)),
            pl.BlockSpec((D, D), lambda i: (0, 0)),
            pl.BlockSpec((1, D), lambda i: (0, 0)),
            pl.BlockSpec((1, D), lambda i: (0, 0)),
            pl.BlockSpec((1, D), lambda i: (0, 0)),
            pl.BlockSpec((D, G + E), lambda i: (0, 0)),
        ],
        out_specs=[
            pl.BlockSpec((rb, D), lambda i: (i, 0)),
            pl.BlockSpec((rb, D), lambda i: (i, 0)),
            pl.BlockSpec((rb, 1), lambda i: (i, 0)),
            pl.BlockSpec((rb, 1), lambda i: (i, 0)),
            pl.BlockSpec((rb, 1), lambda i: (i, 0)),
            pl.BlockSpec((rb, 1), lambda i: (i, 0)),
        ],
        out_shape=[
            jax.ShapeDtypeStruct((nb * rb, D), _F32),
            jax.ShapeDtypeStruct((nb * rb, D), _F32),
            jax.ShapeDtypeStruct((nb * rb, 1), jnp.int32),
            jax.ShapeDtypeStruct((nb * rb, 1), jnp.int32),
            jax.ShapeDtypeStruct((nb * rb, 1), _F32),
            jax.ShapeDtypeStruct((nb * rb, 1), _F32),
        ],
        scratch_shapes=[pltpu.VMEM((8, 128), _F32)],
    )(o, h, wo.astype(jnp.bfloat16), bo.reshape(1, D),
      ln_g.reshape(1, D), ln_b.reshape(1, D), wcat)


def _sc_dispatch(xf, d1, d2):
    mesh = plsc.VectorSubcoreMesh(core_axis_name="c", subcore_axis_name="s",
                                  num_cores=2, num_subcores=16)

    @functools.partial(
        pl.kernel,
        out_type=jax.ShapeDtypeStruct((NBUF, D), _F32),
        mesh=mesh,
        scratch_types=[
            pltpu.VMEM((CHUNK,), jnp.int32),
            pltpu.VMEM((CHUNK,), jnp.int32),
            pltpu.VMEM((CHUNK, D), _F32),
            pltpu.SemaphoreType.DMA,
            pltpu.SemaphoreType.DMA,
            pltpu.SemaphoreType.DMA,
        ],
    )
    def k(xf_hbm, d1_hbm, d2_hbm, buf_hbm, idx1_v, idx2_v, rows_v,
          semr, sem1, sem2):
        wid = lax.axis_index("s") * 2 + lax.axis_index("c")
        base = wid * CHUNK
        cr = pltpu.async_copy(xf_hbm.at[pl.ds(base, CHUNK)], rows_v, semr)
        pltpu.sync_copy(d1_hbm.at[pl.ds(base, CHUNK)], idx1_v)
        pltpu.sync_copy(d2_hbm.at[pl.ds(base, CHUNK)], idx2_v)
        cr.wait()
        c1 = pltpu.async_copy(rows_v, buf_hbm.at[idx1_v], sem1)
        c2 = pltpu.async_copy(rows_v, buf_hbm.at[idx2_v], sem2)
        c1.wait()
        c2.wait()

    return k(xf, d1, d2)


def _sc_gather(y, d1, d2):
    mesh = plsc.VectorSubcoreMesh(core_axis_name="c", subcore_axis_name="s",
                                  num_cores=2, num_subcores=16)

    @functools.partial(
        pl.kernel,
        out_type=(jax.ShapeDtypeStruct((SP, D), _F32),
                  jax.ShapeDtypeStruct((SP, D), _F32)),
        mesh=mesh,
        scratch_types=[
            pltpu.VMEM((CHUNK,), jnp.int32),
            pltpu.VMEM((CHUNK,), jnp.int32),
            pltpu.VMEM((CHUNK, D), _F32),
            pltpu.VMEM((CHUNK, D), _F32),
            pltpu.SemaphoreType.DMA,
            pltpu.SemaphoreType.DMA,
            pltpu.SemaphoreType.DMA,
            pltpu.SemaphoreType.DMA,
        ],
    )
    def k(y_hbm, d1_hbm, d2_hbm, g1_hbm, g2_hbm, idx1_v, idx2_v,
          rows1_v, rows2_v, sem1, sem2, semw1, semw2):
        wid = lax.axis_index("s") * 2 + lax.axis_index("c")
        base = wid * CHUNK
        pltpu.sync_copy(d1_hbm.at[pl.ds(base, CHUNK)], idx1_v)
        pltpu.sync_copy(d2_hbm.at[pl.ds(base, CHUNK)], idx2_v)
        c1 = pltpu.async_copy(y_hbm.at[idx1_v], rows1_v, sem1)
        c2 = pltpu.async_copy(y_hbm.at[idx2_v], rows2_v, sem2)
        c1.wait()
        w1 = pltpu.async_copy(rows1_v, g1_hbm.at[pl.ds(base, CHUNK)], semw1)
        c2.wait()
        w2 = pltpu.async_copy(rows2_v, g2_hbm.at[pl.ds(base, CHUNK)], semw2)
        w1.wait()
        w2.wait()

    return k(y, d1, d2)


def _ffn_kernel(buf_ref, w1_ref, b1_ref, w2_ref, b2_ref, y_ref):
    a = _dotbf(buf_ref[...], w1_ref[0]) + b1_ref[0]
    hgelu = jax.nn.gelu(a)
    y_ref[...] = _dotbf(hgelu, w2_ref[0]) + b2_ref[0]


def _ffn(buf, w1, b1, w2, b2):
    return pl.pallas_call(
        _ffn_kernel,
        grid=(E,),
        in_specs=[
            pl.BlockSpec((CAP, D), lambda e: (e, 0)),
            pl.BlockSpec((1, D, DFF), lambda e: (e, 0, 0)),
            pl.BlockSpec((1, 1, DFF), lambda e: (e, 0, 0)),
            pl.BlockSpec((1, DFF, D), lambda e: (e, 0, 0)),
            pl.BlockSpec((1, 1, D), lambda e: (e, 0, 0)),
        ],
        out_specs=pl.BlockSpec((CAP, D), lambda e: (e, 0)),
        out_shape=jax.ShapeDtypeStruct((NBUF, D), _F32),
    )(buf, w1.astype(jnp.bfloat16), b1.reshape(E, 1, DFF),
      w2.astype(jnp.bfloat16), b2.reshape(E, 1, D))


def _combine_qkv_kernel(h_ref, g1_ref, g2_ref, s1_ref, s2_ref,
                        lg_ref, lb_ref, wq_ref, bias_ref, hc_ref, qkv_ref):
    s1 = s1_ref[...]
    s2 = s2_ref[...]
    t1 = jnp.where(s1 > 0.0, s1 * g1_ref[...], 0.0)
    t2 = jnp.where(s2 > 0.0, s2 * g2_ref[...], 0.0)
    hc = h_ref[...] + t1 + t2
    hc_ref[...] = hc
    t = _ln_rows(hc, lg_ref[...], lb_ref[...])
    qkv_ref[...] = (_dotbf(t, wq_ref[...]) + bias_ref[...]).astype(jnp.bfloat16)


def _combine_qkv(h, gg1, gg2, s1, s2, ln_g, ln_b, wqkv, bqkv):
    return pl.pallas_call(
        _combine_qkv_kernel,
        grid=(4,),
        in_specs=[
            pl.BlockSpec((576, D), lambda i: (i, 0)),
            pl.BlockSpec((576, D), lambda i: (i, 0)),
            pl.BlockSpec((576, D), lambda i: (i, 0)),
            pl.BlockSpec((576, 1), lambda i: (i, 0)),
            pl.BlockSpec((576, 1), lambda i: (i, 0)),
            pl.BlockSpec((1, D), lambda i: (0, 0)),
            pl.BlockSpec((1, D), lambda i: (0, 0)),
            pl.BlockSpec((D, 3 * D), lambda i: (0, 0)),
            pl.BlockSpec((1, 3 * D), lambda i: (0, 0)),
        ],
        out_specs=[
            pl.BlockSpec((576, D), lambda i: (i, 0)),
            pl.BlockSpec((576, 3 * D), lambda i: (i, 0)),
        ],
        out_shape=[
            jax.ShapeDtypeStruct((SP, D), _F32),
            jax.ShapeDtypeStruct((SP, 3 * D), jnp.bfloat16),
        ],
    )(h, gg1, gg2, s1, s2, ln_g.reshape(1, D), ln_b.reshape(1, D),
      wqkv.astype(jnp.bfloat16), bqkv.reshape(1, 3 * D))


def _head_moe_kernel(eids_ref, h0_ref, xf_ref, s_ref, w1_ref, b1_ref,
                     w2_ref, b2_ref, lg_ref, lb_ref, hw_ref, hb_ref,
                     out_ref, acc_ref):
    i = pl.program_id(0)

    @pl.when(i == 0)
    def _init():
        acc_ref[...] = h0_ref[...]

    a = jax.nn.gelu(_dotbf(xf_ref[0:1, :], w1_ref[0]) + b1_ref[0])
    y0 = _dotbf(a, w2_ref[0]) + b2_ref[0]
    sv = s_ref[pl.ds(i, 1), :]
    acc_ref[0:1, :] = acc_ref[0:1, :] + sv * y0

    @pl.when(i == TOPK - 1)
    def _fin():
        rep = _ln_rows(acc_ref[0:1, :], lg_ref[...], lb_ref[...])
        out_ref[...] = _dot32(rep, hw_ref[...]) + hb_ref[...]


def _head_moe(eids, h0, xf0, svec, w1, b1, w2, b2, lnf_g, lnf_b,
              head_w, head_b):
    spec = pltpu.PrefetchScalarGridSpec(
        num_scalar_prefetch=1,
        grid=(TOPK,),
        in_specs=[
            pl.BlockSpec((8, D), lambda i, e: (0, 0)),
            pl.BlockSpec((8, D), lambda i, e: (0, 0)),
            pl.BlockSpec((TOPK, 1), lambda i, e: (0, 0)),
            pl.BlockSpec((1, D, DFF), lambda i, e: (e[i], 0, 0)),
            pl.BlockSpec((1, 1, DFF), lambda i, e: (e[i], 0, 0)),
            pl.BlockSpec((1, DFF, D), lambda i, e: (e[i], 0, 0)),
            pl.BlockSpec((1, 1, D), lambda i, e: (e[i], 0, 0)),
            pl.BlockSpec((1, D), lambda i, e: (0, 0)),
            pl.BlockSpec((1, D), lambda i, e: (0, 0)),
            pl.BlockSpec((D, NCLS), lambda i, e: (0, 0)),
            pl.BlockSpec((1, NCLS), lambda i, e: (0, 0)),
        ],
        out_specs=pl.BlockSpec((1, NCLS), lambda i, e: (0, 0)),
        scratch_shapes=[pltpu.VMEM((8, D), _F32)],
    )
    return pl.pallas_call(
        _head_moe_kernel,
        grid_spec=spec,
        out_shape=jax.ShapeDtypeStruct((1, NCLS), _F32),
    )(eids, h0, xf0, svec, w1.astype(jnp.bfloat16), b1.reshape(E, 1, DFF),
      w2.astype(jnp.bfloat16), b2.reshape(E, 1, D), lnf_g.reshape(1, D),
      lnf_b.reshape(1, D), head_w, head_b.reshape(1, NCLS))


def kernel(x, W_in, b_in, cls_token, pos_emb, ln1_g, ln1_b, Wqkv, bqkv, Wo, bo,
           ln2_g, ln2_b, Wg, We, W1, b1, W2, b2, lnf_g, lnf_b, head_W, head_b):
    xp = jnp.pad(x[0], ((1, SP - 1 - x.shape[1]), (0, 0)))
    table = jnp.concatenate([
        cls_token[0] + pos_emb[0:1],
        pos_emb[1:S1] + b_in[None, :],
        jnp.zeros((SP - S1, D), _F32),
    ], axis=0)

    h, qkv = _embed_qkv(xp, W_in, table, ln1_g[0], ln1_b[0], Wqkv[0], bqkv[0])
    o = _attention(qkv)
    wcat = jnp.concatenate(
        [Wg[0], We[0].transpose(1, 0, 2).reshape(D, E)], axis=1)
    h, xf, d1, d2, s1, s2 = _router(o, h, Wo[0], bo[0],
                                    ln2_g[0], ln2_b[0], wcat)
    d1f = d1.reshape(SP)
    d2f = d2.reshape(SP)
    buf = _sc_dispatch(xf, d1f, d2f)
    y = _ffn(buf, W1[0], b1[0], W2[0], b2[0])
    gg1, gg2 = _sc_gather(y, d1f, d2f)

    h, qkv = _combine_qkv(h, gg1, gg2, s1, s2,
                          ln1_g[1], ln1_b[1], Wqkv[1], bqkv[1])
    o0 = _attention(qkv, nqb=1, qb=128, out_rows=128)
    wcat = jnp.concatenate(
        [Wg[1], We[1].transpose(1, 0, 2).reshape(D, E)], axis=1)
    h0, xf0, d1, d2, s1, s2 = _router(o0, h, Wo[1], bo[1],
                                      ln2_g[1], ln2_b[1], wcat, nb=1, rb=128)
    eids = jnp.stack([d1[0, 0], d2[0, 0]]).astype(jnp.int32) // CAP
    svec = jnp.stack([s1[0, 0], s2[0, 0]]).reshape(TOPK, 1)
    return _head_moe(eids, h0, xf0, svec, W1[1], b1[1], W2[1], b2[1],
                     lnf_g, lnf_b, head_W, head_b)
